# Initial kernel scaffold; baseline (speedup 1.0000x reference)
import jax
import jax.numpy as jnp
from jax.experimental import pallas as pl


def kernel(x, edge_index, pos_edge_index, neg_edge_index, W1, b1, W2, b2, W_ep, b_ep):
    raise NotImplementedError("write your pallas kernel here")


# speedup vs baseline: 23.5212x; 23.5212x over previous
import functools

import jax
import jax.numpy as jnp
from jax import lax
from jax.experimental import pallas as pl
from jax.experimental.pallas import tpu as pltpu
from jax.experimental.pallas import tpu_sc as plsc

_NC = 2
_NS = 16
_NW = _NC * _NS
_CHUNK = 128


def _sc_mesh():
    return plsc.VectorSubcoreMesh(core_axis_name="c", subcore_axis_name="s")


_DEGW = 16


def _deg_kernel(n, e_pad):
    ept = e_pad // _NW
    nchunks = ept // _CHUNK
    n_tbl = ((n + 16 + 127) // 128) * 128
    rows_zero = n_tbl // _NS

    @functools.partial(
        pl.kernel,
        out_type=jax.ShapeDtypeStruct((_NC * n_tbl, _DEGW), jnp.float32),
        mesh=_sc_mesh(),
        compiler_params=pltpu.CompilerParams(use_tc_tiling_on_sc=False),
        scratch_types=[
            pltpu.VMEM((_CHUNK,), jnp.int32),
            pltpu.VMEM((_CHUNK, _DEGW), jnp.float32),
            pltpu.VMEM_SHARED((n_tbl, _DEGW), jnp.float32),
        ],
    )
    def k(dst_hbm, ones_hbm, zeros_hbm, out_hbm, dst_v, ones_v, acc):
        cid = lax.axis_index("c")
        sid = lax.axis_index("s")
        wid = sid * _NC + cid
        pltpu.sync_copy(ones_hbm, ones_v)
        pltpu.sync_copy(zeros_hbm, acc.at[pl.ds(sid * rows_zero, rows_zero)])
        plsc.subcore_barrier()
        base = wid * ept

        def body(j, _):
            pltpu.sync_copy(dst_hbm.at[pl.ds(base + j * _CHUNK, _CHUNK)], dst_v)
            pltpu.sync_copy(ones_v, acc.at[dst_v], add=True)
            return 0

        lax.fori_loop(0, nchunks, body, 0)
        plsc.subcore_barrier()
        pltpu.sync_copy(
            acc.at[pl.ds(sid * rows_zero, rows_zero)],
            out_hbm.at[pl.ds(cid * n_tbl + sid * rows_zero, rows_zero)])

    return k


def _scatter_kernel(n, d, e_pad):
    ept = e_pad // _NW
    nchunks = ept // _CHUNK
    n_tbl = ((n + 16 + 127) // 128) * 128
    rows_zero = n_tbl // _NS

    @functools.partial(
        pl.kernel,
        out_type=jax.ShapeDtypeStruct((_NC * n_tbl, d), jnp.float32),
        mesh=_sc_mesh(),
        compiler_params=pltpu.CompilerParams(use_tc_tiling_on_sc=False),
        scratch_types=[
            pltpu.VMEM((_CHUNK,), jnp.int32),
            pltpu.VMEM((_CHUNK,), jnp.int32),
            pltpu.VMEM((_CHUNK, d), jnp.float32),
            pltpu.VMEM_SHARED((n_tbl, d), jnp.float32),
            pltpu.SemaphoreType.DMA,
        ],
    )
    def k(src_hbm, dst_hbm, g_hbm, zeros_hbm, out_hbm,
          src_v, dst_v, rows_v, acc, sem):
        cid = lax.axis_index("c")
        sid = lax.axis_index("s")
        wid = sid * _NC + cid
        pltpu.sync_copy(zeros_hbm, acc.at[pl.ds(sid * rows_zero, rows_zero)])
        plsc.subcore_barrier()
        base = wid * ept

        def body(j, _):
            off = base + j * _CHUNK
            pltpu.sync_copy(src_hbm.at[pl.ds(off, _CHUNK)], src_v)
            pltpu.sync_copy(dst_hbm.at[pl.ds(off, _CHUNK)], dst_v)
            pltpu.async_copy(g_hbm.at[src_v], rows_v, sem).wait()
            pltpu.sync_copy(rows_v, acc.at[dst_v], add=True)
            return 0

        lax.fori_loop(0, nchunks, body, 0)
        plsc.subcore_barrier()
        pltpu.sync_copy(
            acc.at[pl.ds(sid * rows_zero, rows_zero)],
            out_hbm.at[pl.ds(cid * n_tbl + sid * rows_zero, rows_zero)])

    return k


def _logits_kernel(n, e):
    per = e // _NW
    iters = per // 16

    @functools.partial(
        pl.kernel,
        out_type=jax.ShapeDtypeStruct((2 * e,), jnp.float32),
        mesh=_sc_mesh(),
        compiler_params=pltpu.CompilerParams(needs_layout_passes=False),
        scratch_types=[
            pltpu.VMEM((n,), jnp.float32),
            pltpu.VMEM((n,), jnp.float32),
            pltpu.VMEM((per,), jnp.int32),
            pltpu.VMEM((per,), jnp.int32),
            pltpu.VMEM((per,), jnp.float32),
        ],
    )
    def k(u_hbm, v_hbm, p0_hbm, p1_hbm, n0_hbm, n1_hbm, out_hbm,
          u_v, v_v, a_v, b_v, o_v):
        cid = lax.axis_index("c")
        sid = lax.axis_index("s")
        wid = sid * _NC + cid
        pltpu.sync_copy(u_hbm, u_v)
        pltpu.sync_copy(v_hbm, v_v)
        ebase = wid * per
        for a_hbm, b_hbm, obase in (
                (p0_hbm, p1_hbm, ebase),
                (n0_hbm, n1_hbm, e + ebase)):
            pltpu.sync_copy(a_hbm.at[pl.ds(ebase, per)], a_v)
            pltpu.sync_copy(b_hbm.at[pl.ds(ebase, per)], b_v)

            def body(i, _):
                ia = a_v[pl.ds(i * 16, 16)]
                ib = b_v[pl.ds(i * 16, 16)]
                ga = plsc.load_gather(u_v, [ia])
                gb = plsc.load_gather(v_v, [ib])
                o_v[pl.ds(i * 16, 16)] = ga + gb
                return 0

            lax.fori_loop(0, iters, body, 0)
            pltpu.sync_copy(o_v, out_hbm.at[pl.ds(obase, per)])

    return k


def _tc1(x, w1, degp, n, d_in, d_hid, bn):
    def body(x_ref, w_ref, degp_ref, dinv_ref, g1_ref):
        deg = degp_ref[0][:, 0:1] + degp_ref[1][:, 0:1] + 1.0
        dinv = lax.rsqrt(deg)
        dinv_ref[...] = dinv
        t = jnp.dot(x_ref[...], w_ref[...], preferred_element_type=jnp.float32)
        g1_ref[...] = t * dinv

    grid = n // bn
    return pl.pallas_call(
        body,
        grid=(grid,),
        in_specs=[
            pl.BlockSpec((bn, d_in), lambda i: (i, 0)),
            pl.BlockSpec((d_in, d_hid), lambda i: (0, 0)),
            pl.BlockSpec((_NC, bn, _DEGW), lambda i: (0, i, 0)),
        ],
        out_specs=[
            pl.BlockSpec((bn, 1), lambda i: (i, 0)),
            pl.BlockSpec((bn, d_hid), lambda i: (i, 0)),
        ],
        out_shape=[
            jax.ShapeDtypeStruct((n, 1), jnp.float32),
            jax.ShapeDtypeStruct((n, d_hid), jnp.float32),
        ],
    )(x, w1, degp)


def _tc2(p1, g1, dinv, b1, w2, n, d_hid, d_out, bn):
    def body(p_ref, g1_ref, dinv_ref, b1_ref, w_ref, g2_ref):
        s = p_ref[0] + p_ref[1] + g1_ref[...]
        h = jnp.maximum(s * dinv_ref[...] + b1_ref[...], 0.0)
        t = jnp.dot(h, w_ref[...], preferred_element_type=jnp.float32)
        g2_ref[...] = t * dinv_ref[...]

    grid = n // bn
    return pl.pallas_call(
        body,
        grid=(grid,),
        in_specs=[
            pl.BlockSpec((_NC, bn, d_hid), lambda i: (0, i, 0)),
            pl.BlockSpec((bn, d_hid), lambda i: (i, 0)),
            pl.BlockSpec((bn, 1), lambda i: (i, 0)),
            pl.BlockSpec((1, d_hid), lambda i: (0, 0)),
            pl.BlockSpec((d_hid, d_out), lambda i: (0, 0)),
        ],
        out_specs=pl.BlockSpec((bn, d_out), lambda i: (i, 0)),
        out_shape=jax.ShapeDtypeStruct((n, d_out), jnp.float32),
    )(p1, g1, dinv, b1, w2)


def _tc3(p2, g2, dinv, b2, wep_row, b_ep, n, d_out, bn):
    def body(p_ref, g2_ref, dinv_ref, b2_ref, w_ref, bep_ref,
             z_ref, u_ref, v_ref):
        s = p_ref[0] + p_ref[1] + g2_ref[...]
        z = s * dinv_ref[...] + b2_ref[...]
        z_ref[...] = z
        wa = w_ref[:, 0:d_out]
        wb = w_ref[:, d_out:2 * d_out]
        u_ref[...] = jnp.sum(z * wa, axis=1, keepdims=True) + bep_ref[0, 0]
        v_ref[...] = jnp.sum(z * wb, axis=1, keepdims=True)

    grid = n // bn
    return pl.pallas_call(
        body,
        grid=(grid,),
        in_specs=[
            pl.BlockSpec((_NC, bn, d_out), lambda i: (0, i, 0)),
            pl.BlockSpec((bn, d_out), lambda i: (i, 0)),
            pl.BlockSpec((bn, 1), lambda i: (i, 0)),
            pl.BlockSpec((1, d_out), lambda i: (0, 0)),
            pl.BlockSpec((1, 2 * d_out), lambda i: (0, 0)),
            pl.BlockSpec((1, 1), lambda i: (0, 0)),
        ],
        out_specs=[
            pl.BlockSpec((bn, d_out), lambda i: (i, 0)),
            pl.BlockSpec((bn, 1), lambda i: (i, 0)),
            pl.BlockSpec((bn, 1), lambda i: (i, 0)),
        ],
        out_shape=[
            jax.ShapeDtypeStruct((n, d_out), jnp.float32),
            jax.ShapeDtypeStruct((n, 1), jnp.float32),
            jax.ShapeDtypeStruct((n, 1), jnp.float32),
        ],
    )(p2, g2, dinv, b2, wep_row, b_ep)


def kernel(x, edge_index, pos_edge_index, neg_edge_index,
           W1, b1, W2, b2, W_ep, b_ep):
    n, d_in = x.shape
    e = edge_index.shape[1]
    d_hid = W1.shape[1]
    d_out = W2.shape[1]
    bn = 1000

    ept = -(-e // (_NW * _CHUNK)) * _CHUNK
    e_pad = ept * _NW
    pad = e_pad - e
    e_src = jnp.concatenate([edge_index[0], jnp.zeros((pad,), jnp.int32)])
    e_dst = jnp.concatenate([edge_index[1], jnp.full((pad,), n, jnp.int32)])

    ones_c = jnp.ones((_CHUNK, _DEGW), jnp.float32)
    n_tbl = ((n + 16 + 127) // 128) * 128
    zeros_deg = jnp.zeros((n_tbl // _NS, _DEGW), jnp.float32)
    zeros16 = jnp.zeros((n_tbl // _NS, d_hid), jnp.float32)
    zeros32 = jnp.zeros((n_tbl // _NS, d_out), jnp.float32)

    degp = _deg_kernel(n, e_pad)(e_dst, ones_c, zeros_deg)
    degp = degp.reshape(_NC, n_tbl, _DEGW)

    dinv, g1 = _tc1(x, W1, degp, n, d_in, d_hid, bn)

    p1 = _scatter_kernel(n, d_hid, e_pad)(e_src, e_dst, g1, zeros16)
    p1 = p1.reshape(_NC, n_tbl, d_hid)

    g2 = _tc2(p1, g1, dinv, b1.reshape(1, d_hid), W2, n, d_hid, d_out, bn)

    p2 = _scatter_kernel(n, d_out, e_pad)(e_src, e_dst, g2, zeros32)
    p2 = p2.reshape(_NC, n_tbl, d_out)

    z, u, v = _tc3(p2, g2, dinv, b2.reshape(1, d_out),
                   W_ep.reshape(1, 2 * d_out), b_ep.reshape(1, 1),
                   n, d_out, bn)

    logits = _logits_kernel(n, e)(
        u.reshape(n), v.reshape(n),
        pos_edge_index[0], pos_edge_index[1],
        neg_edge_index[0], neg_edge_index[1])

    return (z, logits.reshape(2 * e, 1))


# speedup vs baseline: 32.7072x; 1.3905x over previous
import functools

import jax
import jax.numpy as jnp
from jax import lax
from jax.experimental import pallas as pl
from jax.experimental.pallas import tpu as pltpu
from jax.experimental.pallas import tpu_sc as plsc

_NC = 2
_NS = 16
_NW = _NC * _NS
_CHUNK = 128
_NBUF = 8


def _sc_mesh():
    return plsc.VectorSubcoreMesh(core_axis_name="c", subcore_axis_name="s")


_DEGW = 16


def _deg_kernel(n, e_pad):
    ept = e_pad // _NW
    nchunks = ept // _CHUNK
    n_tbl = ((n + 16 + 127) // 128) * 128
    rows_zero = n_tbl // _NS

    @functools.partial(
        pl.kernel,
        out_type=jax.ShapeDtypeStruct((_NC * n_tbl, _DEGW), jnp.float32),
        mesh=_sc_mesh(),
        compiler_params=pltpu.CompilerParams(use_tc_tiling_on_sc=False),
        scratch_types=[
            pltpu.VMEM((nchunks, _CHUNK), jnp.int32),
            pltpu.VMEM((_CHUNK, _DEGW), jnp.float32),
            pltpu.VMEM_SHARED((n_tbl, _DEGW), jnp.float32),
            pltpu.SemaphoreType.DMA,
        ],
    )
    def k(dst_hbm, ones_hbm, zeros_hbm, out_hbm, dst_v, ones_v, acc, sem):
        cid = lax.axis_index("c")
        sid = lax.axis_index("s")
        wid = sid * _NC + cid
        pltpu.sync_copy(ones_hbm, ones_v)
        pltpu.sync_copy(dst_hbm.at[pl.ds(wid * nchunks, nchunks)], dst_v)
        pltpu.sync_copy(zeros_hbm, acc.at[pl.ds(sid * rows_zero, rows_zero)])
        plsc.subcore_barrier()

        def body(j, _):
            pltpu.sync_copy(ones_v, acc.at[dst_v.at[j]], add=True)
            return 0

        lax.fori_loop(0, nchunks, body, 0)
        plsc.subcore_barrier()
        pltpu.sync_copy(
            acc.at[pl.ds(sid * rows_zero, rows_zero)],
            out_hbm.at[pl.ds(cid * n_tbl + sid * rows_zero, rows_zero)])

    return k


def _scatter_kernel(n, d, e_pad):
    ept = e_pad // _NW
    nchunks = ept // _CHUNK
    n_tbl = ((n + 16 + 127) // 128) * 128
    rows_zero = n_tbl // _NS

    nbuf = _NBUF

    @functools.partial(
        pl.kernel,
        out_type=jax.ShapeDtypeStruct((_NC * n_tbl, d), jnp.float32),
        mesh=_sc_mesh(),
        compiler_params=pltpu.CompilerParams(use_tc_tiling_on_sc=False),
        scratch_types=[
            pltpu.VMEM((nchunks, _CHUNK), jnp.int32),
        ] + [pltpu.VMEM((_CHUNK,), jnp.int32)] * nbuf
          + [pltpu.VMEM((_CHUNK, d), jnp.float32)] * nbuf + [
            pltpu.VMEM_SHARED((n_tbl, d), jnp.float32),
            pltpu.SemaphoreType.DMA,
            pltpu.SemaphoreType.DMA,
        ],
    )
    def k(src_hbm, dst_hbm, g_hbm, zeros_hbm, out_hbm, dst_v, *rest):
        idx = rest[:nbuf]
        rows = rest[nbuf:2 * nbuf]
        acc = rest[2 * nbuf]
        sem_i = rest[2 * nbuf + 1]
        sem_g = rest[2 * nbuf + 2]
        cid = lax.axis_index("c")
        sid = lax.axis_index("s")
        wid = sid * _NC + cid
        pltpu.sync_copy(dst_hbm.at[pl.ds(wid * nchunks, nchunks)], dst_v)
        pltpu.sync_copy(zeros_hbm, acc.at[pl.ds(sid * rows_zero, rows_zero)])
        plsc.subcore_barrier()
        base = wid * nchunks

        def body(jo, _):
            idescs = [
                pltpu.async_copy(
                    src_hbm.at[base + jo * nbuf + b], idx[b], sem_i)
                for b in range(nbuf)
            ]
            for di in idescs:
                di.wait()
            gdescs = [
                pltpu.async_copy(g_hbm.at[idx[b]], rows[b], sem_g)
                for b in range(nbuf)
            ]
            for dg in gdescs:
                dg.wait()
            for b in range(nbuf):
                pltpu.sync_copy(
                    rows[b], acc.at[dst_v.at[jo * nbuf + b]], add=True)
            return 0

        lax.fori_loop(0, nchunks // nbuf, body, 0)
        plsc.subcore_barrier()
        pltpu.sync_copy(
            acc.at[pl.ds(sid * rows_zero, rows_zero)],
            out_hbm.at[pl.ds(cid * n_tbl + sid * rows_zero, rows_zero)])

    return k


def _logits_kernel(n, e):
    per = e // _NW
    iters = per // 16

    @functools.partial(
        pl.kernel,
        out_type=jax.ShapeDtypeStruct((2 * e,), jnp.float32),
        mesh=_sc_mesh(),
        compiler_params=pltpu.CompilerParams(needs_layout_passes=False),
        scratch_types=[
            pltpu.VMEM((n,), jnp.float32),
            pltpu.VMEM((n,), jnp.float32),
            pltpu.VMEM((per,), jnp.int32),
            pltpu.VMEM((per,), jnp.int32),
            pltpu.VMEM((per,), jnp.float32),
        ],
    )
    def k(u_hbm, v_hbm, p0_hbm, p1_hbm, n0_hbm, n1_hbm, out_hbm,
          u_v, v_v, a_v, b_v, o_v):
        cid = lax.axis_index("c")
        sid = lax.axis_index("s")
        wid = sid * _NC + cid
        pltpu.sync_copy(u_hbm, u_v)
        pltpu.sync_copy(v_hbm, v_v)
        ebase = wid * per
        for a_hbm, b_hbm, obase in (
                (p0_hbm, p1_hbm, ebase),
                (n0_hbm, n1_hbm, e + ebase)):
            pltpu.sync_copy(a_hbm.at[pl.ds(ebase, per)], a_v)
            pltpu.sync_copy(b_hbm.at[pl.ds(ebase, per)], b_v)

            def body(i, _):
                ia = a_v[pl.ds(i * 16, 16)]
                ib = b_v[pl.ds(i * 16, 16)]
                ga = plsc.load_gather(u_v, [ia])
                gb = plsc.load_gather(v_v, [ib])
                o_v[pl.ds(i * 16, 16)] = ga + gb
                return 0

            lax.fori_loop(0, iters, body, 0)
            pltpu.sync_copy(o_v, out_hbm.at[pl.ds(obase, per)])

    return k


def _tc1(x, w1, degp, n, d_in, d_hid, bn):
    def body(x_ref, w_ref, degp_ref, dinv_ref, g1_ref):
        deg = degp_ref[0][:, 0:1] + degp_ref[1][:, 0:1] + 1.0
        dinv = lax.rsqrt(deg)
        dinv_ref[...] = dinv
        t = jnp.dot(x_ref[...], w_ref[...], preferred_element_type=jnp.float32)
        g1_ref[...] = t * dinv

    grid = n // bn
    return pl.pallas_call(
        body,
        grid=(grid,),
        in_specs=[
            pl.BlockSpec((bn, d_in), lambda i: (i, 0)),
            pl.BlockSpec((d_in, d_hid), lambda i: (0, 0)),
            pl.BlockSpec((_NC, bn, _DEGW), lambda i: (0, i, 0)),
        ],
        out_specs=[
            pl.BlockSpec((bn, 1), lambda i: (i, 0)),
            pl.BlockSpec((bn, d_hid), lambda i: (i, 0)),
        ],
        out_shape=[
            jax.ShapeDtypeStruct((n, 1), jnp.float32),
            jax.ShapeDtypeStruct((n, d_hid), jnp.float32),
        ],
    )(x, w1, degp)


def _tc2(p1, g1, dinv, b1, w2, n, d_hid, d_out, bn):
    def body(p_ref, g1_ref, dinv_ref, b1_ref, w_ref, g2_ref):
        s = p_ref[0] + p_ref[1] + g1_ref[...]
        h = jnp.maximum(s * dinv_ref[...] + b1_ref[...], 0.0)
        t = jnp.dot(h, w_ref[...], preferred_element_type=jnp.float32)
        g2_ref[...] = t * dinv_ref[...]

    grid = n // bn
    return pl.pallas_call(
        body,
        grid=(grid,),
        in_specs=[
            pl.BlockSpec((_NC, bn, d_hid), lambda i: (0, i, 0)),
            pl.BlockSpec((bn, d_hid), lambda i: (i, 0)),
            pl.BlockSpec((bn, 1), lambda i: (i, 0)),
            pl.BlockSpec((1, d_hid), lambda i: (0, 0)),
            pl.BlockSpec((d_hid, d_out), lambda i: (0, 0)),
        ],
        out_specs=pl.BlockSpec((bn, d_out), lambda i: (i, 0)),
        out_shape=jax.ShapeDtypeStruct((n, d_out), jnp.float32),
    )(p1, g1, dinv, b1, w2)


def _tc3(p2, g2, dinv, b2, wep_row, b_ep, n, d_out, bn):
    def body(p_ref, g2_ref, dinv_ref, b2_ref, w_ref, bep_ref,
             z_ref, u_ref, v_ref):
        s = p_ref[0] + p_ref[1] + g2_ref[...]
        z = s * dinv_ref[...] + b2_ref[...]
        z_ref[...] = z
        wa = w_ref[:, 0:d_out]
        wb = w_ref[:, d_out:2 * d_out]
        u_ref[...] = jnp.sum(z * wa, axis=1, keepdims=True) + bep_ref[0, 0]
        v_ref[...] = jnp.sum(z * wb, axis=1, keepdims=True)

    grid = n // bn
    return pl.pallas_call(
        body,
        grid=(grid,),
        in_specs=[
            pl.BlockSpec((_NC, bn, d_out), lambda i: (0, i, 0)),
            pl.BlockSpec((bn, d_out), lambda i: (i, 0)),
            pl.BlockSpec((bn, 1), lambda i: (i, 0)),
            pl.BlockSpec((1, d_out), lambda i: (0, 0)),
            pl.BlockSpec((1, 2 * d_out), lambda i: (0, 0)),
            pl.BlockSpec((1, 1), lambda i: (0, 0)),
        ],
        out_specs=[
            pl.BlockSpec((bn, d_out), lambda i: (i, 0)),
            pl.BlockSpec((bn, 1), lambda i: (i, 0)),
            pl.BlockSpec((bn, 1), lambda i: (i, 0)),
        ],
        out_shape=[
            jax.ShapeDtypeStruct((n, d_out), jnp.float32),
            jax.ShapeDtypeStruct((n, 1), jnp.float32),
            jax.ShapeDtypeStruct((n, 1), jnp.float32),
        ],
    )(p2, g2, dinv, b2, wep_row, b_ep)


def kernel(x, edge_index, pos_edge_index, neg_edge_index,
           W1, b1, W2, b2, W_ep, b_ep):
    n, d_in = x.shape
    e = edge_index.shape[1]
    d_hid = W1.shape[1]
    d_out = W2.shape[1]
    bn = 1000

    ept = -(-e // (_NW * _CHUNK * _NBUF)) * (_CHUNK * _NBUF)
    e_pad = ept * _NW
    pad = e_pad - e
    e_src = jnp.concatenate(
        [edge_index[0], jnp.zeros((pad,), jnp.int32)]).reshape(-1, _CHUNK)
    e_dst = jnp.concatenate(
        [edge_index[1], jnp.full((pad,), n, jnp.int32)]).reshape(-1, _CHUNK)

    ones_c = jnp.ones((_CHUNK, _DEGW), jnp.float32)
    n_tbl = ((n + 16 + 127) // 128) * 128
    zeros_deg = jnp.zeros((n_tbl // _NS, _DEGW), jnp.float32)
    zeros16 = jnp.zeros((n_tbl // _NS, d_hid), jnp.float32)
    zeros32 = jnp.zeros((n_tbl // _NS, d_out), jnp.float32)

    degp = _deg_kernel(n, e_pad)(e_dst, ones_c, zeros_deg)
    degp = degp.reshape(_NC, n_tbl, _DEGW)

    dinv, g1 = _tc1(x, W1, degp, n, d_in, d_hid, bn)

    p1 = _scatter_kernel(n, d_hid, e_pad)(e_src, e_dst, g1, zeros16)
    p1 = p1.reshape(_NC, n_tbl, d_hid)

    g2 = _tc2(p1, g1, dinv, b1.reshape(1, d_hid), W2, n, d_hid, d_out, bn)

    p2 = _scatter_kernel(n, d_out, e_pad)(e_src, e_dst, g2, zeros32)
    p2 = p2.reshape(_NC, n_tbl, d_out)

    z, u, v = _tc3(p2, g2, dinv, b2.reshape(1, d_out),
                   W_ep.reshape(1, 2 * d_out), b_ep.reshape(1, 1),
                   n, d_out, bn)

    logits = _logits_kernel(n, e)(
        u.reshape(n), v.reshape(n),
        pos_edge_index[0], pos_edge_index[1],
        neg_edge_index[0], neg_edge_index[1])

    return (z, logits.reshape(2 * e, 1))


# speedup vs baseline: 35.2823x; 1.0787x over previous
import functools

import jax
import jax.numpy as jnp
from jax import lax
from jax.experimental import pallas as pl
from jax.experimental.pallas import tpu as pltpu
from jax.experimental.pallas import tpu_sc as plsc

_NC = 2
_NS = 16
_NW = _NC * _NS
_CHUNK = 128
_NBUF = 16


def _sc_mesh():
    return plsc.VectorSubcoreMesh(core_axis_name="c", subcore_axis_name="s")


_DEGW = 16


def _deg_kernel(n, e_pad):
    ept = e_pad // _NW
    nchunks = ept // _CHUNK
    n_tbl = ((n + 16 + 127) // 128) * 128
    rows_zero = n_tbl // _NS

    @functools.partial(
        pl.kernel,
        out_type=jax.ShapeDtypeStruct((_NC * n_tbl, _DEGW), jnp.float32),
        mesh=_sc_mesh(),
        compiler_params=pltpu.CompilerParams(use_tc_tiling_on_sc=False),
        scratch_types=[
            pltpu.VMEM((nchunks, _CHUNK), jnp.int32),
            pltpu.VMEM((_CHUNK, _DEGW), jnp.float32),
            pltpu.VMEM_SHARED((n_tbl, _DEGW), jnp.float32),
            pltpu.SemaphoreType.DMA,
        ],
    )
    def k(dst_hbm, ones_hbm, zeros_hbm, out_hbm, dst_v, ones_v, acc, sem):
        cid = lax.axis_index("c")
        sid = lax.axis_index("s")
        wid = sid * _NC + cid
        pltpu.sync_copy(ones_hbm, ones_v)
        pltpu.sync_copy(dst_hbm.at[pl.ds(wid * nchunks, nchunks)], dst_v)
        pltpu.sync_copy(zeros_hbm, acc.at[pl.ds(sid * rows_zero, rows_zero)])
        plsc.subcore_barrier()

        def body(j, _):
            pltpu.sync_copy(ones_v, acc.at[dst_v.at[j]], add=True)
            return 0

        lax.fori_loop(0, nchunks, body, 0)
        plsc.subcore_barrier()
        pltpu.sync_copy(
            acc.at[pl.ds(sid * rows_zero, rows_zero)],
            out_hbm.at[pl.ds(cid * n_tbl + sid * rows_zero, rows_zero)])

    return k


def _scatter_kernel(n, d, e_pad):
    ept = e_pad // _NW
    nchunks = ept // _CHUNK
    n_tbl = ((n + 16 + 127) // 128) * 128
    rows_zero = n_tbl // _NS

    nbuf = _NBUF

    @functools.partial(
        pl.kernel,
        out_type=jax.ShapeDtypeStruct((_NC * n_tbl, d), jnp.float32),
        mesh=_sc_mesh(),
        compiler_params=pltpu.CompilerParams(use_tc_tiling_on_sc=False),
        scratch_types=[
            pltpu.VMEM((nchunks, _CHUNK), jnp.int32),
        ] + [pltpu.VMEM((_CHUNK,), jnp.int32)] * nbuf
          + [pltpu.VMEM((_CHUNK, d), jnp.float32)] * nbuf + [
            pltpu.VMEM_SHARED((n_tbl, d), jnp.float32),
            pltpu.SemaphoreType.DMA,
            pltpu.SemaphoreType.DMA,
        ],
    )
    def k(src_hbm, dst_hbm, g_hbm, zeros_hbm, out_hbm, dst_v, *rest):
        idx = rest[:nbuf]
        rows = rest[nbuf:2 * nbuf]
        acc = rest[2 * nbuf]
        sem_i = rest[2 * nbuf + 1]
        sem_g = rest[2 * nbuf + 2]
        cid = lax.axis_index("c")
        sid = lax.axis_index("s")
        wid = sid * _NC + cid
        pltpu.sync_copy(dst_hbm.at[pl.ds(wid * nchunks, nchunks)], dst_v)
        pltpu.sync_copy(zeros_hbm, acc.at[pl.ds(sid * rows_zero, rows_zero)])
        plsc.subcore_barrier()
        base = wid * nchunks

        def body(jo, _):
            idescs = [
                pltpu.async_copy(
                    src_hbm.at[base + jo * nbuf + b], idx[b], sem_i)
                for b in range(nbuf)
            ]
            for di in idescs:
                di.wait()
            gdescs = [
                pltpu.async_copy(g_hbm.at[idx[b]], rows[b], sem_g)
                for b in range(nbuf)
            ]
            for b in range(nbuf):
                gdescs[b].wait()
                pltpu.sync_copy(
                    rows[b], acc.at[dst_v.at[jo * nbuf + b]], add=True)
            return 0

        lax.fori_loop(0, nchunks // nbuf, body, 0)
        plsc.subcore_barrier()
        pltpu.sync_copy(
            acc.at[pl.ds(sid * rows_zero, rows_zero)],
            out_hbm.at[pl.ds(cid * n_tbl + sid * rows_zero, rows_zero)])

    return k


def _logits_kernel(n, e):
    per = e // _NW
    iters = per // 16

    @functools.partial(
        pl.kernel,
        out_type=jax.ShapeDtypeStruct((2 * e,), jnp.float32),
        mesh=_sc_mesh(),
        compiler_params=pltpu.CompilerParams(needs_layout_passes=False),
        scratch_types=[
            pltpu.VMEM((n,), jnp.float32),
            pltpu.VMEM((n,), jnp.float32),
            pltpu.VMEM((per,), jnp.int32),
            pltpu.VMEM((per,), jnp.int32),
            pltpu.VMEM((per,), jnp.float32),
        ],
    )
    def k(u_hbm, v_hbm, p0_hbm, p1_hbm, n0_hbm, n1_hbm, out_hbm,
          u_v, v_v, a_v, b_v, o_v):
        cid = lax.axis_index("c")
        sid = lax.axis_index("s")
        wid = sid * _NC + cid
        pltpu.sync_copy(u_hbm, u_v)
        pltpu.sync_copy(v_hbm, v_v)
        ebase = wid * per
        for a_hbm, b_hbm, obase in (
                (p0_hbm, p1_hbm, ebase),
                (n0_hbm, n1_hbm, e + ebase)):
            pltpu.sync_copy(a_hbm.at[pl.ds(ebase, per)], a_v)
            pltpu.sync_copy(b_hbm.at[pl.ds(ebase, per)], b_v)

            def body(i, _):
                ia = a_v[pl.ds(i * 16, 16)]
                ib = b_v[pl.ds(i * 16, 16)]
                ga = plsc.load_gather(u_v, [ia])
                gb = plsc.load_gather(v_v, [ib])
                o_v[pl.ds(i * 16, 16)] = ga + gb
                return 0

            lax.fori_loop(0, iters, body, 0)
            pltpu.sync_copy(o_v, out_hbm.at[pl.ds(obase, per)])

    return k


def _tc1(x, w1, degp, n, d_in, d_hid, bn):
    def body(x_ref, w_ref, degp_ref, dinv_ref, g1_ref):
        deg = degp_ref[0][:, 0:1] + degp_ref[1][:, 0:1] + 1.0
        dinv = lax.rsqrt(deg)
        dinv_ref[...] = dinv
        t = jnp.dot(x_ref[...], w_ref[...], preferred_element_type=jnp.float32)
        g1_ref[...] = t * dinv

    grid = n // bn
    return pl.pallas_call(
        body,
        grid=(grid,),
        in_specs=[
            pl.BlockSpec((bn, d_in), lambda i: (i, 0)),
            pl.BlockSpec((d_in, d_hid), lambda i: (0, 0)),
            pl.BlockSpec((_NC, bn, _DEGW), lambda i: (0, i, 0)),
        ],
        out_specs=[
            pl.BlockSpec((bn, 1), lambda i: (i, 0)),
            pl.BlockSpec((bn, d_hid), lambda i: (i, 0)),
        ],
        out_shape=[
            jax.ShapeDtypeStruct((n, 1), jnp.float32),
            jax.ShapeDtypeStruct((n, d_hid), jnp.float32),
        ],
    )(x, w1, degp)


def _tc2(p1, g1, dinv, b1, w2, n, d_hid, d_out, bn):
    def body(p_ref, g1_ref, dinv_ref, b1_ref, w_ref, g2_ref):
        s = p_ref[0] + p_ref[1] + g1_ref[...]
        h = jnp.maximum(s * dinv_ref[...] + b1_ref[...], 0.0)
        t = jnp.dot(h, w_ref[...], preferred_element_type=jnp.float32)
        g2_ref[...] = t * dinv_ref[...]

    grid = n // bn
    return pl.pallas_call(
        body,
        grid=(grid,),
        in_specs=[
            pl.BlockSpec((_NC, bn, d_hid), lambda i: (0, i, 0)),
            pl.BlockSpec((bn, d_hid), lambda i: (i, 0)),
            pl.BlockSpec((bn, 1), lambda i: (i, 0)),
            pl.BlockSpec((1, d_hid), lambda i: (0, 0)),
            pl.BlockSpec((d_hid, d_out), lambda i: (0, 0)),
        ],
        out_specs=pl.BlockSpec((bn, d_out), lambda i: (i, 0)),
        out_shape=jax.ShapeDtypeStruct((n, d_out), jnp.float32),
    )(p1, g1, dinv, b1, w2)


def _tc3(p2, g2, dinv, b2, wep_row, b_ep, n, d_out, bn):
    def body(p_ref, g2_ref, dinv_ref, b2_ref, w_ref, bep_ref,
             z_ref, u_ref, v_ref):
        s = p_ref[0] + p_ref[1] + g2_ref[...]
        z = s * dinv_ref[...] + b2_ref[...]
        z_ref[...] = z
        wa = w_ref[:, 0:d_out]
        wb = w_ref[:, d_out:2 * d_out]
        u_ref[...] = jnp.sum(z * wa, axis=1, keepdims=True) + bep_ref[0, 0]
        v_ref[...] = jnp.sum(z * wb, axis=1, keepdims=True)

    grid = n // bn
    return pl.pallas_call(
        body,
        grid=(grid,),
        in_specs=[
            pl.BlockSpec((_NC, bn, d_out), lambda i: (0, i, 0)),
            pl.BlockSpec((bn, d_out), lambda i: (i, 0)),
            pl.BlockSpec((bn, 1), lambda i: (i, 0)),
            pl.BlockSpec((1, d_out), lambda i: (0, 0)),
            pl.BlockSpec((1, 2 * d_out), lambda i: (0, 0)),
            pl.BlockSpec((1, 1), lambda i: (0, 0)),
        ],
        out_specs=[
            pl.BlockSpec((bn, d_out), lambda i: (i, 0)),
            pl.BlockSpec((bn, 1), lambda i: (i, 0)),
            pl.BlockSpec((bn, 1), lambda i: (i, 0)),
        ],
        out_shape=[
            jax.ShapeDtypeStruct((n, d_out), jnp.float32),
            jax.ShapeDtypeStruct((n, 1), jnp.float32),
            jax.ShapeDtypeStruct((n, 1), jnp.float32),
        ],
    )(p2, g2, dinv, b2, wep_row, b_ep)


def kernel(x, edge_index, pos_edge_index, neg_edge_index,
           W1, b1, W2, b2, W_ep, b_ep):
    n, d_in = x.shape
    e = edge_index.shape[1]
    d_hid = W1.shape[1]
    d_out = W2.shape[1]
    bn = 1000

    ept = -(-e // (_NW * _CHUNK * _NBUF)) * (_CHUNK * _NBUF)
    e_pad = ept * _NW
    pad = e_pad - e
    e_src = jnp.concatenate(
        [edge_index[0], jnp.zeros((pad,), jnp.int32)]).reshape(-1, _CHUNK)
    e_dst = jnp.concatenate(
        [edge_index[1], jnp.full((pad,), n, jnp.int32)]).reshape(-1, _CHUNK)

    ones_c = jnp.ones((_CHUNK, _DEGW), jnp.float32)
    n_tbl = ((n + 16 + 127) // 128) * 128
    zeros_deg = jnp.zeros((n_tbl // _NS, _DEGW), jnp.float32)
    zeros16 = jnp.zeros((n_tbl // _NS, d_hid), jnp.float32)
    zeros32 = jnp.zeros((n_tbl // _NS, d_out), jnp.float32)

    degp = _deg_kernel(n, e_pad)(e_dst, ones_c, zeros_deg)
    degp = degp.reshape(_NC, n_tbl, _DEGW)

    dinv, g1 = _tc1(x, W1, degp, n, d_in, d_hid, bn)

    p1 = _scatter_kernel(n, d_hid, e_pad)(e_src, e_dst, g1, zeros16)
    p1 = p1.reshape(_NC, n_tbl, d_hid)

    g2 = _tc2(p1, g1, dinv, b1.reshape(1, d_hid), W2, n, d_hid, d_out, bn)

    p2 = _scatter_kernel(n, d_out, e_pad)(e_src, e_dst, g2, zeros32)
    p2 = p2.reshape(_NC, n_tbl, d_out)

    z, u, v = _tc3(p2, g2, dinv, b2.reshape(1, d_out),
                   W_ep.reshape(1, 2 * d_out), b_ep.reshape(1, 1),
                   n, d_out, bn)

    logits = _logits_kernel(n, e)(
        u.reshape(n), v.reshape(n),
        pos_edge_index[0], pos_edge_index[1],
        neg_edge_index[0], neg_edge_index[1])

    return (z, logits.reshape(2 * e, 1))


# speedup vs baseline: 35.9860x; 1.0199x over previous
import functools

import jax
import jax.numpy as jnp
from jax import lax
from jax.experimental import pallas as pl
from jax.experimental.pallas import tpu as pltpu
from jax.experimental.pallas import tpu_sc as plsc

_NC = 2
_NS = 16
_NW = _NC * _NS
_CHUNK = 128
_NBUF = 16


def _sc_mesh():
    return plsc.VectorSubcoreMesh(core_axis_name="c", subcore_axis_name="s")


_DEGW = 16


def _deg_kernel(n, e_pad):
    ept = e_pad // _NW
    nchunks = ept // _CHUNK
    n_tbl = ((n + 16 + 127) // 128) * 128
    rows_zero = n_tbl // _NS

    @functools.partial(
        pl.kernel,
        out_type=jax.ShapeDtypeStruct((_NC * n_tbl, _DEGW), jnp.float32),
        mesh=_sc_mesh(),
        compiler_params=pltpu.CompilerParams(use_tc_tiling_on_sc=False),
        scratch_types=[
            pltpu.VMEM((nchunks, _CHUNK), jnp.int32),
            pltpu.VMEM((_CHUNK, _DEGW), jnp.float32),
            pltpu.VMEM_SHARED((n_tbl, _DEGW), jnp.float32),
            pltpu.SemaphoreType.DMA,
        ],
    )
    def k(dst_hbm, ones_hbm, zeros_hbm, out_hbm, dst_v, ones_v, acc, sem):
        cid = lax.axis_index("c")
        sid = lax.axis_index("s")
        wid = sid * _NC + cid
        pltpu.sync_copy(ones_hbm, ones_v)
        pltpu.sync_copy(dst_hbm.at[pl.ds(wid * nchunks, nchunks)], dst_v)
        pltpu.sync_copy(zeros_hbm, acc.at[pl.ds(sid * rows_zero, rows_zero)])
        plsc.subcore_barrier()

        def body(j, _):
            pltpu.sync_copy(ones_v, acc.at[dst_v.at[j]], add=True)
            return 0

        lax.fori_loop(0, nchunks, body, 0)
        plsc.subcore_barrier()
        pltpu.sync_copy(
            acc.at[pl.ds(sid * rows_zero, rows_zero)],
            out_hbm.at[pl.ds(cid * n_tbl + sid * rows_zero, rows_zero)])

    return k


def _scatter_kernel(n, d, e_pad):
    ept = e_pad // _NW
    nchunks = ept // _CHUNK
    n_tbl = ((n + 16 + 127) // 128) * 128
    rows_zero = n_tbl // _NS

    nbuf = _NBUF

    @functools.partial(
        pl.kernel,
        out_type=jax.ShapeDtypeStruct((_NC * n_tbl, d), jnp.float32),
        mesh=_sc_mesh(),
        compiler_params=pltpu.CompilerParams(use_tc_tiling_on_sc=False),
        scratch_types=[
            pltpu.VMEM((nchunks, _CHUNK), jnp.int32),
            pltpu.VMEM((nbuf, _CHUNK), jnp.int32),
        ] + [pltpu.VMEM((_CHUNK, d), jnp.float32)] * nbuf + [
            pltpu.VMEM_SHARED((n_tbl, d), jnp.float32),
            pltpu.SemaphoreType.DMA,
        ],
    )
    def k(src_hbm, dst_hbm, g_hbm, zeros_hbm, out_hbm, dst_v, idxw, *rest):
        rows = rest[:nbuf]
        acc = rest[nbuf]
        sem_g = rest[nbuf + 1]
        cid = lax.axis_index("c")
        sid = lax.axis_index("s")
        wid = sid * _NC + cid
        pltpu.sync_copy(dst_hbm.at[pl.ds(wid * nchunks, nchunks)], dst_v)
        pltpu.sync_copy(zeros_hbm, acc.at[pl.ds(sid * rows_zero, rows_zero)])
        plsc.subcore_barrier()
        base = wid * nchunks

        def body(jo, _):
            pltpu.sync_copy(
                src_hbm.at[pl.ds(base + jo * nbuf, nbuf)], idxw)
            gdescs = [
                pltpu.async_copy(g_hbm.at[idxw.at[b]], rows[b], sem_g)
                for b in range(nbuf)
            ]
            for b in range(nbuf):
                gdescs[b].wait()
                pltpu.sync_copy(
                    rows[b], acc.at[dst_v.at[jo * nbuf + b]], add=True)
            return 0

        lax.fori_loop(0, nchunks // nbuf, body, 0)
        plsc.subcore_barrier()
        pltpu.sync_copy(
            acc.at[pl.ds(sid * rows_zero, rows_zero)],
            out_hbm.at[pl.ds(cid * n_tbl + sid * rows_zero, rows_zero)])

    return k


def _logits_kernel(n, e):
    per = e // _NW
    iters = per // 16

    @functools.partial(
        pl.kernel,
        out_type=jax.ShapeDtypeStruct((2 * e,), jnp.float32),
        mesh=_sc_mesh(),
        compiler_params=pltpu.CompilerParams(needs_layout_passes=False),
        scratch_types=[
            pltpu.VMEM((n,), jnp.float32),
            pltpu.VMEM((n,), jnp.float32),
            pltpu.VMEM((per,), jnp.int32),
            pltpu.VMEM((per,), jnp.int32),
            pltpu.VMEM((per,), jnp.float32),
        ],
    )
    def k(u_hbm, v_hbm, p0_hbm, p1_hbm, n0_hbm, n1_hbm, out_hbm,
          u_v, v_v, a_v, b_v, o_v):
        cid = lax.axis_index("c")
        sid = lax.axis_index("s")
        wid = sid * _NC + cid
        pltpu.sync_copy(u_hbm, u_v)
        pltpu.sync_copy(v_hbm, v_v)
        ebase = wid * per
        for a_hbm, b_hbm, obase in (
                (p0_hbm, p1_hbm, ebase),
                (n0_hbm, n1_hbm, e + ebase)):
            pltpu.sync_copy(a_hbm.at[pl.ds(ebase, per)], a_v)
            pltpu.sync_copy(b_hbm.at[pl.ds(ebase, per)], b_v)

            def body(i, _):
                ia = a_v[pl.ds(i * 16, 16)]
                ib = b_v[pl.ds(i * 16, 16)]
                ga = plsc.load_gather(u_v, [ia])
                gb = plsc.load_gather(v_v, [ib])
                o_v[pl.ds(i * 16, 16)] = ga + gb
                return 0

            lax.fori_loop(0, iters, body, 0)
            pltpu.sync_copy(o_v, out_hbm.at[pl.ds(obase, per)])

    return k


def _tc1(x, w1, degp, n, d_in, d_hid, bn):
    def body(x_ref, w_ref, degp_ref, dinv_ref, g1_ref):
        deg = degp_ref[0][:, 0:1] + degp_ref[1][:, 0:1] + 1.0
        dinv = lax.rsqrt(deg)
        dinv_ref[...] = dinv
        t = jnp.dot(x_ref[...], w_ref[...], preferred_element_type=jnp.float32)
        g1_ref[...] = t * dinv

    grid = n // bn
    return pl.pallas_call(
        body,
        grid=(grid,),
        in_specs=[
            pl.BlockSpec((bn, d_in), lambda i: (i, 0)),
            pl.BlockSpec((d_in, d_hid), lambda i: (0, 0)),
            pl.BlockSpec((_NC, bn, _DEGW), lambda i: (0, i, 0)),
        ],
        out_specs=[
            pl.BlockSpec((bn, 1), lambda i: (i, 0)),
            pl.BlockSpec((bn, d_hid), lambda i: (i, 0)),
        ],
        out_shape=[
            jax.ShapeDtypeStruct((n, 1), jnp.float32),
            jax.ShapeDtypeStruct((n, d_hid), jnp.float32),
        ],
    )(x, w1, degp)


def _tc2(p1, g1, dinv, b1, w2, n, d_hid, d_out, bn):
    def body(p_ref, g1_ref, dinv_ref, b1_ref, w_ref, g2_ref):
        s = p_ref[0] + p_ref[1] + g1_ref[...]
        h = jnp.maximum(s * dinv_ref[...] + b1_ref[...], 0.0)
        t = jnp.dot(h, w_ref[...], preferred_element_type=jnp.float32)
        g2_ref[...] = t * dinv_ref[...]

    grid = n // bn
    return pl.pallas_call(
        body,
        grid=(grid,),
        in_specs=[
            pl.BlockSpec((_NC, bn, d_hid), lambda i: (0, i, 0)),
            pl.BlockSpec((bn, d_hid), lambda i: (i, 0)),
            pl.BlockSpec((bn, 1), lambda i: (i, 0)),
            pl.BlockSpec((1, d_hid), lambda i: (0, 0)),
            pl.BlockSpec((d_hid, d_out), lambda i: (0, 0)),
        ],
        out_specs=pl.BlockSpec((bn, d_out), lambda i: (i, 0)),
        out_shape=jax.ShapeDtypeStruct((n, d_out), jnp.float32),
    )(p1, g1, dinv, b1, w2)


def _tc3(p2, g2, dinv, b2, wep_row, b_ep, n, d_out, bn):
    def body(p_ref, g2_ref, dinv_ref, b2_ref, w_ref, bep_ref,
             z_ref, u_ref, v_ref):
        s = p_ref[0] + p_ref[1] + g2_ref[...]
        z = s * dinv_ref[...] + b2_ref[...]
        z_ref[...] = z
        wa = w_ref[:, 0:d_out]
        wb = w_ref[:, d_out:2 * d_out]
        u_ref[...] = jnp.sum(z * wa, axis=1, keepdims=True) + bep_ref[0, 0]
        v_ref[...] = jnp.sum(z * wb, axis=1, keepdims=True)

    grid = n // bn
    return pl.pallas_call(
        body,
        grid=(grid,),
        in_specs=[
            pl.BlockSpec((_NC, bn, d_out), lambda i: (0, i, 0)),
            pl.BlockSpec((bn, d_out), lambda i: (i, 0)),
            pl.BlockSpec((bn, 1), lambda i: (i, 0)),
            pl.BlockSpec((1, d_out), lambda i: (0, 0)),
            pl.BlockSpec((1, 2 * d_out), lambda i: (0, 0)),
            pl.BlockSpec((1, 1), lambda i: (0, 0)),
        ],
        out_specs=[
            pl.BlockSpec((bn, d_out), lambda i: (i, 0)),
            pl.BlockSpec((bn, 1), lambda i: (i, 0)),
            pl.BlockSpec((bn, 1), lambda i: (i, 0)),
        ],
        out_shape=[
            jax.ShapeDtypeStruct((n, d_out), jnp.float32),
            jax.ShapeDtypeStruct((n, 1), jnp.float32),
            jax.ShapeDtypeStruct((n, 1), jnp.float32),
        ],
    )(p2, g2, dinv, b2, wep_row, b_ep)


def kernel(x, edge_index, pos_edge_index, neg_edge_index,
           W1, b1, W2, b2, W_ep, b_ep):
    n, d_in = x.shape
    e = edge_index.shape[1]
    d_hid = W1.shape[1]
    d_out = W2.shape[1]
    bn = 1000

    ept = -(-e // (_NW * _CHUNK * _NBUF)) * (_CHUNK * _NBUF)
    e_pad = ept * _NW
    pad = e_pad - e
    e_src = jnp.concatenate(
        [edge_index[0], jnp.zeros((pad,), jnp.int32)]).reshape(-1, _CHUNK)
    e_dst = jnp.concatenate(
        [edge_index[1], jnp.full((pad,), n, jnp.int32)]).reshape(-1, _CHUNK)

    ones_c = jnp.ones((_CHUNK, _DEGW), jnp.float32)
    n_tbl = ((n + 16 + 127) // 128) * 128
    zeros_deg = jnp.zeros((n_tbl // _NS, _DEGW), jnp.float32)
    zeros16 = jnp.zeros((n_tbl // _NS, d_hid), jnp.float32)
    zeros32 = jnp.zeros((n_tbl // _NS, d_out), jnp.float32)

    degp = _deg_kernel(n, e_pad)(e_dst, ones_c, zeros_deg)
    degp = degp.reshape(_NC, n_tbl, _DEGW)

    dinv, g1 = _tc1(x, W1, degp, n, d_in, d_hid, bn)

    p1 = _scatter_kernel(n, d_hid, e_pad)(e_src, e_dst, g1, zeros16)
    p1 = p1.reshape(_NC, n_tbl, d_hid)

    g2 = _tc2(p1, g1, dinv, b1.reshape(1, d_hid), W2, n, d_hid, d_out, bn)

    p2 = _scatter_kernel(n, d_out, e_pad)(e_src, e_dst, g2, zeros32)
    p2 = p2.reshape(_NC, n_tbl, d_out)

    z, u, v = _tc3(p2, g2, dinv, b2.reshape(1, d_out),
                   W_ep.reshape(1, 2 * d_out), b_ep.reshape(1, 1),
                   n, d_out, bn)

    logits = _logits_kernel(n, e)(
        u.reshape(n), v.reshape(n),
        pos_edge_index[0], pos_edge_index[1],
        neg_edge_index[0], neg_edge_index[1])

    return (z, logits.reshape(2 * e, 1))


# speedup vs baseline: 52.1265x; 1.4485x over previous
import functools

import jax
import jax.numpy as jnp
from jax import lax
from jax.experimental import pallas as pl
from jax.experimental.pallas import tpu as pltpu
from jax.experimental.pallas import tpu_sc as plsc

_NC = 2
_NS = 16
_NW = _NC * _NS
_CHUNK = 128
_NBUF = 16


def _sc_mesh():
    return plsc.VectorSubcoreMesh(core_axis_name="c", subcore_axis_name="s")


_DEGW = 16


def _deg_kernel(n, e_pad):
    ept = e_pad // _NW
    nchunks = ept // _CHUNK
    n_tbl = ((n + 16 + 127) // 128) * 128
    rows_zero = n_tbl // _NS

    @functools.partial(
        pl.kernel,
        out_type=jax.ShapeDtypeStruct((_NC * n_tbl, _DEGW), jnp.float32),
        mesh=_sc_mesh(),
        compiler_params=pltpu.CompilerParams(use_tc_tiling_on_sc=False),
        scratch_types=[
            pltpu.VMEM((nchunks, _CHUNK), jnp.int32),
            pltpu.VMEM((_CHUNK, _DEGW), jnp.float32),
            pltpu.VMEM_SHARED((n_tbl, _DEGW), jnp.float32),
            pltpu.SemaphoreType.DMA,
        ],
    )
    def k(dst_hbm, ones_hbm, zeros_hbm, out_hbm, dst_v, ones_v, acc, sem):
        cid = lax.axis_index("c")
        sid = lax.axis_index("s")
        wid = sid * _NC + cid
        pltpu.sync_copy(ones_hbm, ones_v)
        pltpu.sync_copy(dst_hbm.at[pl.ds(wid * nchunks, nchunks)], dst_v)
        pltpu.sync_copy(zeros_hbm, acc.at[pl.ds(sid * rows_zero, rows_zero)])
        plsc.subcore_barrier()

        def body(j, _):
            pltpu.sync_copy(ones_v, acc.at[dst_v.at[j]], add=True)
            return 0

        lax.fori_loop(0, nchunks, body, 0)
        plsc.subcore_barrier()
        pltpu.sync_copy(
            acc.at[pl.ds(sid * rows_zero, rows_zero)],
            out_hbm.at[pl.ds(cid * n_tbl + sid * rows_zero, rows_zero)])

    return k


def _scatter_kernel(n, d, e_pad):
    ept = e_pad // _NW
    nchunks = ept // _CHUNK
    n_tbl = ((n + 16 + 127) // 128) * 128
    rows_zero = n_tbl // _NS

    nbuf = _NBUF

    rows_stage = n // _NS

    @functools.partial(
        pl.kernel,
        out_type=jax.ShapeDtypeStruct((_NC * n_tbl, d), jnp.float32),
        mesh=_sc_mesh(),
        compiler_params=pltpu.CompilerParams(use_tc_tiling_on_sc=False),
        scratch_types=[
            pltpu.VMEM((nchunks, _CHUNK), jnp.int32),
            pltpu.VMEM((nbuf, _CHUNK), jnp.int32),
        ] + [pltpu.VMEM((_CHUNK, d), jnp.float32)] * nbuf + [
            pltpu.VMEM_SHARED((n_tbl, d), jnp.float32),
            pltpu.VMEM_SHARED((n, d), jnp.float32),
            pltpu.SemaphoreType.DMA,
        ],
    )
    def k(src_hbm, dst_hbm, g_hbm, zeros_hbm, out_hbm, dst_v, idxw, *rest):
        rows = rest[:nbuf]
        acc = rest[nbuf]
        g_sp = rest[nbuf + 1]
        sem_g = rest[nbuf + 2]
        cid = lax.axis_index("c")
        sid = lax.axis_index("s")
        wid = sid * _NC + cid
        pltpu.sync_copy(dst_hbm.at[pl.ds(wid * nchunks, nchunks)], dst_v)
        pltpu.sync_copy(
            g_hbm.at[pl.ds(sid * rows_stage, rows_stage)],
            g_sp.at[pl.ds(sid * rows_stage, rows_stage)])
        pltpu.sync_copy(zeros_hbm, acc.at[pl.ds(sid * rows_zero, rows_zero)])
        plsc.subcore_barrier()
        base = wid * nchunks

        def body(jo, _):
            pltpu.sync_copy(
                src_hbm.at[pl.ds(base + jo * nbuf, nbuf)], idxw)
            gdescs = [
                pltpu.async_copy(g_sp.at[idxw.at[b]], rows[b], sem_g)
                for b in range(nbuf)
            ]
            for b in range(nbuf):
                gdescs[b].wait()
                pltpu.sync_copy(
                    rows[b], acc.at[dst_v.at[jo * nbuf + b]], add=True)
            return 0

        lax.fori_loop(0, nchunks // nbuf, body, 0)
        plsc.subcore_barrier()
        pltpu.sync_copy(
            acc.at[pl.ds(sid * rows_zero, rows_zero)],
            out_hbm.at[pl.ds(cid * n_tbl + sid * rows_zero, rows_zero)])

    return k


def _logits_kernel(n, e):
    per = e // _NW
    iters = per // 16

    @functools.partial(
        pl.kernel,
        out_type=jax.ShapeDtypeStruct((2 * e,), jnp.float32),
        mesh=_sc_mesh(),
        compiler_params=pltpu.CompilerParams(needs_layout_passes=False),
        scratch_types=[
            pltpu.VMEM((n,), jnp.float32),
            pltpu.VMEM((n,), jnp.float32),
            pltpu.VMEM((per,), jnp.int32),
            pltpu.VMEM((per,), jnp.int32),
            pltpu.VMEM((per,), jnp.float32),
        ],
    )
    def k(u_hbm, v_hbm, p0_hbm, p1_hbm, n0_hbm, n1_hbm, out_hbm,
          u_v, v_v, a_v, b_v, o_v):
        cid = lax.axis_index("c")
        sid = lax.axis_index("s")
        wid = sid * _NC + cid
        pltpu.sync_copy(u_hbm, u_v)
        pltpu.sync_copy(v_hbm, v_v)
        ebase = wid * per
        for a_hbm, b_hbm, obase in (
                (p0_hbm, p1_hbm, ebase),
                (n0_hbm, n1_hbm, e + ebase)):
            pltpu.sync_copy(a_hbm.at[pl.ds(ebase, per)], a_v)
            pltpu.sync_copy(b_hbm.at[pl.ds(ebase, per)], b_v)

            def body(i, _):
                ia = a_v[pl.ds(i * 16, 16)]
                ib = b_v[pl.ds(i * 16, 16)]
                ga = plsc.load_gather(u_v, [ia])
                gb = plsc.load_gather(v_v, [ib])
                o_v[pl.ds(i * 16, 16)] = ga + gb
                return 0

            lax.fori_loop(0, iters, body, 0)
            pltpu.sync_copy(o_v, out_hbm.at[pl.ds(obase, per)])

    return k


def _tc1(x, w1, degp, n, d_in, d_hid, bn):
    def body(x_ref, w_ref, degp_ref, dinv_ref, g1_ref):
        deg = degp_ref[0][:, 0:1] + degp_ref[1][:, 0:1] + 1.0
        dinv = lax.rsqrt(deg)
        dinv_ref[...] = dinv
        t = jnp.dot(x_ref[...], w_ref[...], preferred_element_type=jnp.float32)
        g1_ref[...] = t * dinv

    grid = n // bn
    return pl.pallas_call(
        body,
        grid=(grid,),
        in_specs=[
            pl.BlockSpec((bn, d_in), lambda i: (i, 0)),
            pl.BlockSpec((d_in, d_hid), lambda i: (0, 0)),
            pl.BlockSpec((_NC, bn, _DEGW), lambda i: (0, i, 0)),
        ],
        out_specs=[
            pl.BlockSpec((bn, 1), lambda i: (i, 0)),
            pl.BlockSpec((bn, d_hid), lambda i: (i, 0)),
        ],
        out_shape=[
            jax.ShapeDtypeStruct((n, 1), jnp.float32),
            jax.ShapeDtypeStruct((n, d_hid), jnp.float32),
        ],
    )(x, w1, degp)


def _tc2(p1, g1, dinv, b1, w2, n, d_hid, d_out, bn):
    def body(p_ref, g1_ref, dinv_ref, b1_ref, w_ref, g2_ref):
        s = p_ref[0] + p_ref[1] + g1_ref[...]
        h = jnp.maximum(s * dinv_ref[...] + b1_ref[...], 0.0)
        t = jnp.dot(h, w_ref[...], preferred_element_type=jnp.float32)
        g2_ref[...] = t * dinv_ref[...]

    grid = n // bn
    return pl.pallas_call(
        body,
        grid=(grid,),
        in_specs=[
            pl.BlockSpec((_NC, bn, d_hid), lambda i: (0, i, 0)),
            pl.BlockSpec((bn, d_hid), lambda i: (i, 0)),
            pl.BlockSpec((bn, 1), lambda i: (i, 0)),
            pl.BlockSpec((1, d_hid), lambda i: (0, 0)),
            pl.BlockSpec((d_hid, d_out), lambda i: (0, 0)),
        ],
        out_specs=pl.BlockSpec((bn, d_out), lambda i: (i, 0)),
        out_shape=jax.ShapeDtypeStruct((n, d_out), jnp.float32),
    )(p1, g1, dinv, b1, w2)


def _tc3(p2, g2, dinv, b2, wep_row, b_ep, n, d_out, bn):
    def body(p_ref, g2_ref, dinv_ref, b2_ref, w_ref, bep_ref,
             z_ref, u_ref, v_ref):
        s = p_ref[0] + p_ref[1] + g2_ref[...]
        z = s * dinv_ref[...] + b2_ref[...]
        z_ref[...] = z
        wa = w_ref[:, 0:d_out]
        wb = w_ref[:, d_out:2 * d_out]
        u_ref[...] = jnp.sum(z * wa, axis=1, keepdims=True) + bep_ref[0, 0]
        v_ref[...] = jnp.sum(z * wb, axis=1, keepdims=True)

    grid = n // bn
    return pl.pallas_call(
        body,
        grid=(grid,),
        in_specs=[
            pl.BlockSpec((_NC, bn, d_out), lambda i: (0, i, 0)),
            pl.BlockSpec((bn, d_out), lambda i: (i, 0)),
            pl.BlockSpec((bn, 1), lambda i: (i, 0)),
            pl.BlockSpec((1, d_out), lambda i: (0, 0)),
            pl.BlockSpec((1, 2 * d_out), lambda i: (0, 0)),
            pl.BlockSpec((1, 1), lambda i: (0, 0)),
        ],
        out_specs=[
            pl.BlockSpec((bn, d_out), lambda i: (i, 0)),
            pl.BlockSpec((bn, 1), lambda i: (i, 0)),
            pl.BlockSpec((bn, 1), lambda i: (i, 0)),
        ],
        out_shape=[
            jax.ShapeDtypeStruct((n, d_out), jnp.float32),
            jax.ShapeDtypeStruct((n, 1), jnp.float32),
            jax.ShapeDtypeStruct((n, 1), jnp.float32),
        ],
    )(p2, g2, dinv, b2, wep_row, b_ep)


def kernel(x, edge_index, pos_edge_index, neg_edge_index,
           W1, b1, W2, b2, W_ep, b_ep):
    n, d_in = x.shape
    e = edge_index.shape[1]
    d_hid = W1.shape[1]
    d_out = W2.shape[1]
    bn = 1000

    ept = -(-e // (_NW * _CHUNK * _NBUF)) * (_CHUNK * _NBUF)
    e_pad = ept * _NW
    pad = e_pad - e
    e_src = jnp.concatenate(
        [edge_index[0], jnp.zeros((pad,), jnp.int32)]).reshape(-1, _CHUNK)
    e_dst = jnp.concatenate(
        [edge_index[1], jnp.full((pad,), n, jnp.int32)]).reshape(-1, _CHUNK)

    ones_c = jnp.ones((_CHUNK, _DEGW), jnp.float32)
    n_tbl = ((n + 16 + 127) // 128) * 128
    zeros_deg = jnp.zeros((n_tbl // _NS, _DEGW), jnp.float32)
    zeros16 = jnp.zeros((n_tbl // _NS, d_hid), jnp.float32)
    zeros32 = jnp.zeros((n_tbl // _NS, d_out), jnp.float32)

    degp = _deg_kernel(n, e_pad)(e_dst, ones_c, zeros_deg)
    degp = degp.reshape(_NC, n_tbl, _DEGW)

    dinv, g1 = _tc1(x, W1, degp, n, d_in, d_hid, bn)

    p1 = _scatter_kernel(n, d_hid, e_pad)(e_src, e_dst, g1, zeros16)
    p1 = p1.reshape(_NC, n_tbl, d_hid)

    g2 = _tc2(p1, g1, dinv, b1.reshape(1, d_hid), W2, n, d_hid, d_out, bn)

    p2 = _scatter_kernel(n, d_out, e_pad)(e_src, e_dst, g2, zeros32)
    p2 = p2.reshape(_NC, n_tbl, d_out)

    z, u, v = _tc3(p2, g2, dinv, b2.reshape(1, d_out),
                   W_ep.reshape(1, 2 * d_out), b_ep.reshape(1, 1),
                   n, d_out, bn)

    logits = _logits_kernel(n, e)(
        u.reshape(n), v.reshape(n),
        pos_edge_index[0], pos_edge_index[1],
        neg_edge_index[0], neg_edge_index[1])

    return (z, logits.reshape(2 * e, 1))


# speedup vs baseline: 52.1318x; 1.0001x over previous
import functools

import jax
import jax.numpy as jnp
from jax import lax
from jax.experimental import pallas as pl
from jax.experimental.pallas import tpu as pltpu
from jax.experimental.pallas import tpu_sc as plsc

_NC = 2
_NS = 16
_NW = _NC * _NS
_CHUNK = 128
_NBUF = 16


def _sc_mesh():
    return plsc.VectorSubcoreMesh(core_axis_name="c", subcore_axis_name="s")


_DEGW = 16


def _deg_kernel(n, e_pad):
    ept = e_pad // _NW
    nchunks = ept // _CHUNK
    n_tbl = ((n + 16 + 127) // 128) * 128
    rows_zero = n_tbl // _NS

    @functools.partial(
        pl.kernel,
        out_type=jax.ShapeDtypeStruct((_NC * n_tbl, _DEGW), jnp.float32),
        mesh=_sc_mesh(),
        compiler_params=pltpu.CompilerParams(use_tc_tiling_on_sc=False),
        scratch_types=[
            pltpu.VMEM((nchunks, _CHUNK), jnp.int32),
            pltpu.VMEM((_CHUNK, _DEGW), jnp.float32),
            pltpu.VMEM_SHARED((n_tbl, _DEGW), jnp.float32),
            pltpu.SemaphoreType.DMA,
        ],
    )
    def k(dst_hbm, ones_hbm, zeros_hbm, out_hbm, dst_v, ones_v, acc, sem):
        cid = lax.axis_index("c")
        sid = lax.axis_index("s")
        wid = sid * _NC + cid
        pltpu.sync_copy(ones_hbm, ones_v)
        pltpu.sync_copy(dst_hbm.at[pl.ds(wid * nchunks, nchunks)], dst_v)
        pltpu.sync_copy(zeros_hbm, acc.at[pl.ds(sid * rows_zero, rows_zero)])
        plsc.subcore_barrier()

        def body(j, _):
            pltpu.sync_copy(ones_v, acc.at[dst_v.at[j]], add=True)
            return 0

        lax.fori_loop(0, nchunks, body, 0)
        plsc.subcore_barrier()
        pltpu.sync_copy(
            acc.at[pl.ds(sid * rows_zero, rows_zero)],
            out_hbm.at[pl.ds(cid * n_tbl + sid * rows_zero, rows_zero)])

    return k


def _scatter_kernel(n, d, e_pad):
    ept = e_pad // _NW
    nchunks = ept // _CHUNK
    n_tbl = ((n + 16 + 127) // 128) * 128
    rows_zero = n_tbl // _NS

    nbuf = _NBUF

    rows_stage = n // _NS

    @functools.partial(
        pl.kernel,
        out_type=jax.ShapeDtypeStruct((_NC * n_tbl, d), jnp.float32),
        mesh=_sc_mesh(),
        compiler_params=pltpu.CompilerParams(use_tc_tiling_on_sc=False),
        scratch_types=[
            pltpu.VMEM((nchunks, _CHUNK), jnp.int32),
            pltpu.VMEM((nbuf, _CHUNK), jnp.int32),
        ] + [pltpu.VMEM((_CHUNK, d), jnp.float32)] * nbuf + [
            pltpu.VMEM_SHARED((n_tbl, d), jnp.float32),
            pltpu.VMEM_SHARED((n, d), jnp.float32),
            pltpu.SemaphoreType.DMA,
        ],
    )
    def k(src_hbm, dst_hbm, g_hbm, zeros_hbm, out_hbm, dst_v, idxw, *rest):
        rows = rest[:nbuf]
        acc = rest[nbuf]
        g_sp = rest[nbuf + 1]
        sem_g = rest[nbuf + 2]
        cid = lax.axis_index("c")
        sid = lax.axis_index("s")
        wid = sid * _NC + cid
        pltpu.sync_copy(dst_hbm.at[pl.ds(wid * nchunks, nchunks)], dst_v)
        pltpu.sync_copy(
            g_hbm.at[pl.ds(sid * rows_stage, rows_stage)],
            g_sp.at[pl.ds(sid * rows_stage, rows_stage)])
        pltpu.sync_copy(zeros_hbm, acc.at[pl.ds(sid * rows_zero, rows_zero)])
        plsc.subcore_barrier()
        base = wid * nchunks

        def body(jo, _):
            pltpu.sync_copy(
                src_hbm.at[pl.ds(base + jo * nbuf, nbuf)], idxw)
            gdescs = [
                pltpu.async_copy(g_sp.at[idxw.at[b]], rows[b], sem_g)
                for b in range(nbuf)
            ]
            for b in range(nbuf):
                gdescs[b].wait()
                pltpu.sync_copy(
                    rows[b], acc.at[dst_v.at[jo * nbuf + b]], add=True)
            return 0

        lax.fori_loop(0, nchunks // nbuf, body, 0)
        plsc.subcore_barrier()
        pltpu.sync_copy(
            acc.at[pl.ds(sid * rows_zero, rows_zero)],
            out_hbm.at[pl.ds(cid * n_tbl + sid * rows_zero, rows_zero)])

    return k


def _logits_kernel(n, e):
    per = e // _NW
    iters = per // 16

    @functools.partial(
        pl.kernel,
        out_type=jax.ShapeDtypeStruct((2 * e,), jnp.float32),
        mesh=_sc_mesh(),
        compiler_params=pltpu.CompilerParams(needs_layout_passes=False),
        scratch_types=[
            pltpu.VMEM((n,), jnp.float32),
            pltpu.VMEM((n,), jnp.float32),
            pltpu.VMEM((per,), jnp.int32),
            pltpu.VMEM((per,), jnp.int32),
            pltpu.VMEM((per,), jnp.float32),
        ],
    )
    def k(u_hbm, v_hbm, p0_hbm, p1_hbm, n0_hbm, n1_hbm, out_hbm,
          u_v, v_v, a_v, b_v, o_v):
        cid = lax.axis_index("c")
        sid = lax.axis_index("s")
        wid = sid * _NC + cid
        pltpu.sync_copy(u_hbm, u_v)
        pltpu.sync_copy(v_hbm, v_v)
        ebase = wid * per
        for a_hbm, b_hbm, obase in (
                (p0_hbm, p1_hbm, ebase),
                (n0_hbm, n1_hbm, e + ebase)):
            pltpu.sync_copy(a_hbm.at[pl.ds(ebase, per)], a_v)
            pltpu.sync_copy(b_hbm.at[pl.ds(ebase, per)], b_v)

            def body(i, _):
                ia = a_v[pl.ds(i * 16, 16)]
                ib = b_v[pl.ds(i * 16, 16)]
                ga = plsc.load_gather(u_v, [ia])
                gb = plsc.load_gather(v_v, [ib])
                o_v[pl.ds(i * 16, 16)] = ga + gb
                return 0

            lax.fori_loop(0, iters, body, 0)
            pltpu.sync_copy(o_v, out_hbm.at[pl.ds(obase, per)])

    return k


def _tc0(x, w1, n, d_in, d_hid, bn):
    def body(x_ref, w_ref, t_ref):
        t_ref[...] = jnp.dot(
            x_ref[...], w_ref[...], preferred_element_type=jnp.float32)

    grid = n // bn
    return pl.pallas_call(
        body,
        grid=(grid,),
        in_specs=[
            pl.BlockSpec((bn, d_in), lambda i: (i, 0)),
            pl.BlockSpec((d_in, d_hid), lambda i: (0, 0)),
        ],
        out_specs=pl.BlockSpec((bn, d_hid), lambda i: (i, 0)),
        out_shape=jax.ShapeDtypeStruct((n, d_hid), jnp.float32),
    )(x, w1)


def _tc1(t1, degp, n, d_hid, bn):
    def body(t_ref, degp_ref, dinv_ref, g1_ref):
        deg = degp_ref[0][:, 0:1] + degp_ref[1][:, 0:1] + 1.0
        dinv = lax.rsqrt(deg)
        dinv_ref[...] = dinv
        g1_ref[...] = t_ref[...] * dinv

    grid = n // bn
    return pl.pallas_call(
        body,
        grid=(grid,),
        in_specs=[
            pl.BlockSpec((bn, d_hid), lambda i: (i, 0)),
            pl.BlockSpec((_NC, bn, _DEGW), lambda i: (0, i, 0)),
        ],
        out_specs=[
            pl.BlockSpec((bn, 1), lambda i: (i, 0)),
            pl.BlockSpec((bn, d_hid), lambda i: (i, 0)),
        ],
        out_shape=[
            jax.ShapeDtypeStruct((n, 1), jnp.float32),
            jax.ShapeDtypeStruct((n, d_hid), jnp.float32),
        ],
    )(t1, degp)


def _tc2(p1, g1, dinv, b1, w2, n, d_hid, d_out, bn):
    def body(p_ref, g1_ref, dinv_ref, b1_ref, w_ref, g2_ref):
        s = p_ref[0] + p_ref[1] + g1_ref[...]
        h = jnp.maximum(s * dinv_ref[...] + b1_ref[...], 0.0)
        t = jnp.dot(h, w_ref[...], preferred_element_type=jnp.float32)
        g2_ref[...] = t * dinv_ref[...]

    grid = n // bn
    return pl.pallas_call(
        body,
        grid=(grid,),
        in_specs=[
            pl.BlockSpec((_NC, bn, d_hid), lambda i: (0, i, 0)),
            pl.BlockSpec((bn, d_hid), lambda i: (i, 0)),
            pl.BlockSpec((bn, 1), lambda i: (i, 0)),
            pl.BlockSpec((1, d_hid), lambda i: (0, 0)),
            pl.BlockSpec((d_hid, d_out), lambda i: (0, 0)),
        ],
        out_specs=pl.BlockSpec((bn, d_out), lambda i: (i, 0)),
        out_shape=jax.ShapeDtypeStruct((n, d_out), jnp.float32),
    )(p1, g1, dinv, b1, w2)


def _tc3(p2, g2, dinv, b2, wep_row, b_ep, n, d_out, bn):
    def body(p_ref, g2_ref, dinv_ref, b2_ref, w_ref, bep_ref,
             z_ref, u_ref, v_ref):
        s = p_ref[0] + p_ref[1] + g2_ref[...]
        z = s * dinv_ref[...] + b2_ref[...]
        z_ref[...] = z
        wa = w_ref[:, 0:d_out]
        wb = w_ref[:, d_out:2 * d_out]
        u_ref[...] = jnp.sum(z * wa, axis=1, keepdims=True) + bep_ref[0, 0]
        v_ref[...] = jnp.sum(z * wb, axis=1, keepdims=True)

    grid = n // bn
    return pl.pallas_call(
        body,
        grid=(grid,),
        in_specs=[
            pl.BlockSpec((_NC, bn, d_out), lambda i: (0, i, 0)),
            pl.BlockSpec((bn, d_out), lambda i: (i, 0)),
            pl.BlockSpec((bn, 1), lambda i: (i, 0)),
            pl.BlockSpec((1, d_out), lambda i: (0, 0)),
            pl.BlockSpec((1, 2 * d_out), lambda i: (0, 0)),
            pl.BlockSpec((1, 1), lambda i: (0, 0)),
        ],
        out_specs=[
            pl.BlockSpec((bn, d_out), lambda i: (i, 0)),
            pl.BlockSpec((bn, 1), lambda i: (i, 0)),
            pl.BlockSpec((bn, 1), lambda i: (i, 0)),
        ],
        out_shape=[
            jax.ShapeDtypeStruct((n, d_out), jnp.float32),
            jax.ShapeDtypeStruct((n, 1), jnp.float32),
            jax.ShapeDtypeStruct((n, 1), jnp.float32),
        ],
    )(p2, g2, dinv, b2, wep_row, b_ep)


def kernel(x, edge_index, pos_edge_index, neg_edge_index,
           W1, b1, W2, b2, W_ep, b_ep):
    n, d_in = x.shape
    e = edge_index.shape[1]
    d_hid = W1.shape[1]
    d_out = W2.shape[1]
    bn = 1000

    ept = -(-e // (_NW * _CHUNK * _NBUF)) * (_CHUNK * _NBUF)
    e_pad = ept * _NW
    pad = e_pad - e
    e_src = jnp.concatenate(
        [edge_index[0], jnp.zeros((pad,), jnp.int32)]).reshape(-1, _CHUNK)
    e_dst = jnp.concatenate(
        [edge_index[1], jnp.full((pad,), n, jnp.int32)]).reshape(-1, _CHUNK)

    ones_c = jnp.ones((_CHUNK, _DEGW), jnp.float32)
    n_tbl = ((n + 16 + 127) // 128) * 128
    zeros_deg = jnp.zeros((n_tbl // _NS, _DEGW), jnp.float32)
    zeros16 = jnp.zeros((n_tbl // _NS, d_hid), jnp.float32)
    zeros32 = jnp.zeros((n_tbl // _NS, d_out), jnp.float32)

    t1 = _tc0(x, W1, n, d_in, d_hid, bn)
    degp = _deg_kernel(n, e_pad)(e_dst, ones_c, zeros_deg)
    degp = degp.reshape(_NC, n_tbl, _DEGW)

    dinv, g1 = _tc1(t1, degp, n, d_hid, bn)

    p1 = _scatter_kernel(n, d_hid, e_pad)(e_src, e_dst, g1, zeros16)
    p1 = p1.reshape(_NC, n_tbl, d_hid)

    g2 = _tc2(p1, g1, dinv, b1.reshape(1, d_hid), W2, n, d_hid, d_out, bn)

    p2 = _scatter_kernel(n, d_out, e_pad)(e_src, e_dst, g2, zeros32)
    p2 = p2.reshape(_NC, n_tbl, d_out)

    z, u, v = _tc3(p2, g2, dinv, b2.reshape(1, d_out),
                   W_ep.reshape(1, 2 * d_out), b_ep.reshape(1, 1),
                   n, d_out, bn)

    logits = _logits_kernel(n, e)(
        u.reshape(n), v.reshape(n),
        pos_edge_index[0], pos_edge_index[1],
        neg_edge_index[0], neg_edge_index[1])

    return (z, logits.reshape(2 * e, 1))


# speedup vs baseline: 52.3301x; 1.0038x over previous
import functools

import jax
import jax.numpy as jnp
from jax import lax
from jax.experimental import pallas as pl
from jax.experimental.pallas import tpu as pltpu
from jax.experimental.pallas import tpu_sc as plsc

_NC = 2
_NS = 16
_NW = _NC * _NS
_CHUNK = 128
_NBUF = 16


def _sc_mesh():
    return plsc.VectorSubcoreMesh(core_axis_name="c", subcore_axis_name="s")


_DEGW = 16


def _deg_kernel(n, e_pad):
    ept = e_pad // _NW
    nchunks = ept // _CHUNK
    n_tbl = ((n + 16 + 127) // 128) * 128
    rows_zero = n_tbl // _NS

    @functools.partial(
        pl.kernel,
        out_type=jax.ShapeDtypeStruct((_NC * n_tbl, _DEGW), jnp.float32),
        mesh=_sc_mesh(),
        compiler_params=pltpu.CompilerParams(use_tc_tiling_on_sc=False),
        scratch_types=[
            pltpu.VMEM((nchunks, _CHUNK), jnp.int32),
            pltpu.VMEM((_CHUNK, _DEGW), jnp.float32),
            pltpu.VMEM_SHARED((n_tbl, _DEGW), jnp.float32),
            pltpu.SemaphoreType.DMA,
        ],
    )
    def k(dst_hbm, ones_hbm, zeros_hbm, out_hbm, dst_v, ones_v, acc, sem):
        cid = lax.axis_index("c")
        sid = lax.axis_index("s")
        wid = sid * _NC + cid
        pltpu.sync_copy(ones_hbm, ones_v)
        pltpu.sync_copy(dst_hbm.at[pl.ds(wid * nchunks, nchunks)], dst_v)
        pltpu.sync_copy(zeros_hbm, acc.at[pl.ds(sid * rows_zero, rows_zero)])
        plsc.subcore_barrier()

        def body(j, _):
            pltpu.sync_copy(ones_v, acc.at[dst_v.at[j]], add=True)
            return 0

        lax.fori_loop(0, nchunks, body, 0)
        plsc.subcore_barrier()
        pltpu.sync_copy(
            acc.at[pl.ds(sid * rows_zero, rows_zero)],
            out_hbm.at[pl.ds(cid * n_tbl + sid * rows_zero, rows_zero)])

    return k


def _scatter_kernel(n, d, e_pad):
    ept = e_pad // _NW
    nchunks = ept // _CHUNK
    n_tbl = ((n + 16 + 127) // 128) * 128
    rows_zero = n_tbl // _NS

    nbuf = _NBUF

    rows_stage = n // _NS

    @functools.partial(
        pl.kernel,
        out_type=jax.ShapeDtypeStruct((_NC * n_tbl, d), jnp.float32),
        mesh=_sc_mesh(),
        compiler_params=pltpu.CompilerParams(use_tc_tiling_on_sc=False),
        scratch_types=[
            pltpu.VMEM((nchunks, _CHUNK), jnp.int32),
            pltpu.VMEM((nbuf, _CHUNK), jnp.int32),
        ] + [pltpu.VMEM((_CHUNK, d), jnp.float32)] * nbuf + [
            pltpu.VMEM_SHARED((n_tbl, d), jnp.float32),
            pltpu.VMEM_SHARED((n, d), jnp.float32),
            pltpu.SemaphoreType.DMA,
        ],
    )
    def k(src_hbm, dst_hbm, g_hbm, zeros_hbm, out_hbm, dst_v, idxw, *rest):
        rows = rest[:nbuf]
        acc = rest[nbuf]
        g_sp = rest[nbuf + 1]
        sem_g = rest[nbuf + 2]
        cid = lax.axis_index("c")
        sid = lax.axis_index("s")
        wid = sid * _NC + cid
        pltpu.sync_copy(dst_hbm.at[pl.ds(wid * nchunks, nchunks)], dst_v)
        pltpu.sync_copy(
            g_hbm.at[pl.ds(sid * rows_stage, rows_stage)],
            g_sp.at[pl.ds(sid * rows_stage, rows_stage)])
        pltpu.sync_copy(zeros_hbm, acc.at[pl.ds(sid * rows_zero, rows_zero)])
        plsc.subcore_barrier()
        base = wid * nchunks

        def body(jo, _):
            pltpu.sync_copy(
                src_hbm.at[pl.ds(base + jo * nbuf, nbuf)], idxw)
            gdescs = [
                pltpu.async_copy(g_sp.at[idxw.at[b]], rows[b], sem_g)
                for b in range(nbuf)
            ]
            for b in range(nbuf):
                gdescs[b].wait()
                pltpu.sync_copy(
                    rows[b], acc.at[dst_v.at[jo * nbuf + b]], add=True)
            return 0

        lax.fori_loop(0, nchunks // nbuf, body, 0)
        plsc.subcore_barrier()
        pltpu.sync_copy(
            acc.at[pl.ds(sid * rows_zero, rows_zero)],
            out_hbm.at[pl.ds(cid * n_tbl + sid * rows_zero, rows_zero)])

    return k


def _logits_kernel(n, e):
    per = e // _NW
    iters = per // 16

    @functools.partial(
        pl.kernel,
        out_type=jax.ShapeDtypeStruct((2 * e,), jnp.float32),
        mesh=_sc_mesh(),
        compiler_params=pltpu.CompilerParams(needs_layout_passes=False),
        scratch_types=[
            pltpu.VMEM((n,), jnp.float32),
            pltpu.VMEM((n,), jnp.float32),
            pltpu.VMEM((per,), jnp.int32),
            pltpu.VMEM((per,), jnp.int32),
            pltpu.VMEM((per,), jnp.float32),
        ],
    )
    def k(u_hbm, v_hbm, p0_hbm, p1_hbm, n0_hbm, n1_hbm, out_hbm,
          u_v, v_v, a_v, b_v, o_v):
        cid = lax.axis_index("c")
        sid = lax.axis_index("s")
        wid = sid * _NC + cid
        pltpu.sync_copy(u_hbm, u_v)
        pltpu.sync_copy(v_hbm, v_v)
        ebase = wid * per
        for a_hbm, b_hbm, obase in (
                (p0_hbm, p1_hbm, ebase),
                (n0_hbm, n1_hbm, e + ebase)):
            pltpu.sync_copy(a_hbm.at[pl.ds(ebase, per)], a_v)
            pltpu.sync_copy(b_hbm.at[pl.ds(ebase, per)], b_v)

            def body(i, _):
                ia = a_v[pl.ds(i * 16, 16)]
                ib = b_v[pl.ds(i * 16, 16)]
                ga = plsc.load_gather(u_v, [ia])
                gb = plsc.load_gather(v_v, [ib])
                o_v[pl.ds(i * 16, 16)] = ga + gb
                return 0

            lax.fori_loop(0, iters, body, 0)
            pltpu.sync_copy(o_v, out_hbm.at[pl.ds(obase, per)])

    return k


def _tc1(x, w1, degp, n, d_in, d_hid, bn):
    def body(x_ref, w_ref, degp_ref, dinv_ref, g1_ref):
        deg = degp_ref[0][:, 0:1] + degp_ref[1][:, 0:1] + 1.0
        dinv = lax.rsqrt(deg)
        dinv_ref[...] = dinv
        t = jnp.dot(x_ref[...], w_ref[...], preferred_element_type=jnp.float32)
        g1_ref[...] = t * dinv

    grid = n // bn
    return pl.pallas_call(
        body,
        grid=(grid,),
        in_specs=[
            pl.BlockSpec((bn, d_in), lambda i: (i, 0)),
            pl.BlockSpec((d_in, d_hid), lambda i: (0, 0)),
            pl.BlockSpec((_NC, bn, _DEGW), lambda i: (0, i, 0)),
        ],
        out_specs=[
            pl.BlockSpec((bn, 1), lambda i: (i, 0)),
            pl.BlockSpec((bn, d_hid), lambda i: (i, 0)),
        ],
        out_shape=[
            jax.ShapeDtypeStruct((n, 1), jnp.float32),
            jax.ShapeDtypeStruct((n, d_hid), jnp.float32),
        ],
    )(x, w1, degp)


def _tc2(p1, g1, dinv, b1, w2, n, d_hid, d_out, bn):
    def body(p_ref, g1_ref, dinv_ref, b1_ref, w_ref, g2_ref):
        s = p_ref[0] + p_ref[1] + g1_ref[...]
        h = jnp.maximum(s * dinv_ref[...] + b1_ref[...], 0.0)
        t = jnp.dot(h, w_ref[...], preferred_element_type=jnp.float32)
        g2_ref[...] = t * dinv_ref[...]

    grid = n // bn
    return pl.pallas_call(
        body,
        grid=(grid,),
        in_specs=[
            pl.BlockSpec((_NC, bn, d_hid), lambda i: (0, i, 0)),
            pl.BlockSpec((bn, d_hid), lambda i: (i, 0)),
            pl.BlockSpec((bn, 1), lambda i: (i, 0)),
            pl.BlockSpec((1, d_hid), lambda i: (0, 0)),
            pl.BlockSpec((d_hid, d_out), lambda i: (0, 0)),
        ],
        out_specs=pl.BlockSpec((bn, d_out), lambda i: (i, 0)),
        out_shape=jax.ShapeDtypeStruct((n, d_out), jnp.float32),
    )(p1, g1, dinv, b1, w2)


def _tc3(p2, g2, dinv, b2, wep_row, b_ep, n, d_out, bn):
    def body(p_ref, g2_ref, dinv_ref, b2_ref, w_ref, bep_ref,
             z_ref, u_ref, v_ref):
        s = p_ref[0] + p_ref[1] + g2_ref[...]
        z = s * dinv_ref[...] + b2_ref[...]
        z_ref[...] = z
        wa = w_ref[:, 0:d_out]
        wb = w_ref[:, d_out:2 * d_out]
        u_ref[...] = jnp.sum(z * wa, axis=1, keepdims=True) + bep_ref[0, 0]
        v_ref[...] = jnp.sum(z * wb, axis=1, keepdims=True)

    grid = n // bn
    return pl.pallas_call(
        body,
        grid=(grid,),
        in_specs=[
            pl.BlockSpec((_NC, bn, d_out), lambda i: (0, i, 0)),
            pl.BlockSpec((bn, d_out), lambda i: (i, 0)),
            pl.BlockSpec((bn, 1), lambda i: (i, 0)),
            pl.BlockSpec((1, d_out), lambda i: (0, 0)),
            pl.BlockSpec((1, 2 * d_out), lambda i: (0, 0)),
            pl.BlockSpec((1, 1), lambda i: (0, 0)),
        ],
        out_specs=[
            pl.BlockSpec((bn, d_out), lambda i: (i, 0)),
            pl.BlockSpec((bn, 1), lambda i: (i, 0)),
            pl.BlockSpec((bn, 1), lambda i: (i, 0)),
        ],
        out_shape=[
            jax.ShapeDtypeStruct((n, d_out), jnp.float32),
            jax.ShapeDtypeStruct((n, 1), jnp.float32),
            jax.ShapeDtypeStruct((n, 1), jnp.float32),
        ],
    )(p2, g2, dinv, b2, wep_row, b_ep)


def kernel(x, edge_index, pos_edge_index, neg_edge_index,
           W1, b1, W2, b2, W_ep, b_ep):
    n, d_in = x.shape
    e = edge_index.shape[1]
    d_hid = W1.shape[1]
    d_out = W2.shape[1]
    bn = 1000

    ept = -(-e // (_NW * _CHUNK * _NBUF)) * (_CHUNK * _NBUF)
    e_pad = ept * _NW
    pad = e_pad - e
    e_src = jnp.concatenate(
        [edge_index[0], jnp.zeros((pad,), jnp.int32)]).reshape(-1, _CHUNK)
    e_dst = jnp.concatenate(
        [edge_index[1], jnp.full((pad,), n, jnp.int32)]).reshape(-1, _CHUNK)

    ones_c = jnp.ones((_CHUNK, _DEGW), jnp.float32)
    n_tbl = ((n + 16 + 127) // 128) * 128
    zeros_deg = jnp.zeros((n_tbl // _NS, _DEGW), jnp.float32)
    zeros16 = jnp.zeros((n_tbl // _NS, d_hid), jnp.float32)
    zeros32 = jnp.zeros((n_tbl // _NS, d_out), jnp.float32)

    degp = _deg_kernel(n, e_pad)(e_dst, ones_c, zeros_deg)
    degp = degp.reshape(_NC, n_tbl, _DEGW)

    dinv, g1 = _tc1(x, W1, degp, n, d_in, d_hid, bn)

    p1 = _scatter_kernel(n, d_hid, e_pad)(e_src, e_dst, g1, zeros16)
    p1 = p1.reshape(_NC, n_tbl, d_hid)

    g2 = _tc2(p1, g1, dinv, b1.reshape(1, d_hid), W2, n, d_hid, d_out, bn)

    p2 = _scatter_kernel(n, d_out, e_pad)(e_src, e_dst, g2, zeros32)
    p2 = p2.reshape(_NC, n_tbl, d_out)

    z, u, v = _tc3(p2, g2, dinv, b2.reshape(1, d_out),
                   W_ep.reshape(1, 2 * d_out), b_ep.reshape(1, 1),
                   n, d_out, bn)

    logits = _logits_kernel(n, e)(
        u.reshape(n), v.reshape(n),
        pos_edge_index[0], pos_edge_index[1],
        neg_edge_index[0], neg_edge_index[1])

    return (z, logits.reshape(2 * e, 1))


# speedup vs baseline: 52.9149x; 1.0112x over previous
import functools

import jax
import jax.numpy as jnp
from jax import lax
from jax.experimental import pallas as pl
from jax.experimental.pallas import tpu as pltpu
from jax.experimental.pallas import tpu_sc as plsc

_NC = 2
_NS = 16
_NW = _NC * _NS
_CHUNK = 128
_NBUF = 16


def _sc_mesh():
    return plsc.VectorSubcoreMesh(core_axis_name="c", subcore_axis_name="s")


_DEGW = 16


def _deg_kernel(n, e_pad):
    ept = e_pad // _NW
    nchunks = ept // _CHUNK
    n_tbl = ((n + 16 + 127) // 128) * 128
    rows_zero = n_tbl // _NS

    @functools.partial(
        pl.kernel,
        out_type=jax.ShapeDtypeStruct((_NC * n_tbl, _DEGW), jnp.float32),
        mesh=_sc_mesh(),
        compiler_params=pltpu.CompilerParams(use_tc_tiling_on_sc=False),
        scratch_types=[
            pltpu.VMEM((nchunks, _CHUNK), jnp.int32),
            pltpu.VMEM((_CHUNK, _DEGW), jnp.float32),
            pltpu.VMEM_SHARED((n_tbl, _DEGW), jnp.float32),
            pltpu.SemaphoreType.DMA,
        ],
    )
    def k(dst_hbm, ones_hbm, zeros_hbm, out_hbm, dst_v, ones_v, acc, sem):
        cid = lax.axis_index("c")
        sid = lax.axis_index("s")
        wid = sid * _NC + cid
        pltpu.sync_copy(ones_hbm, ones_v)
        pltpu.sync_copy(dst_hbm.at[pl.ds(wid * nchunks, nchunks)], dst_v)
        pltpu.sync_copy(zeros_hbm, acc.at[pl.ds(sid * rows_zero, rows_zero)])
        plsc.subcore_barrier()

        dwin = 16

        def body(jo, _):
            descs = [
                pltpu.async_copy(
                    ones_v, acc.at[dst_v.at[jo * dwin + b]], sem, add=True)
                for b in range(dwin)
            ]
            for dsc in descs:
                dsc.wait()
            return 0

        lax.fori_loop(0, nchunks // dwin, body, 0)

        def tail(j, _):
            pltpu.sync_copy(ones_v, acc.at[dst_v.at[j]], add=True)
            return 0

        lax.fori_loop((nchunks // dwin) * dwin, nchunks, tail, 0)
        plsc.subcore_barrier()
        pltpu.sync_copy(
            acc.at[pl.ds(sid * rows_zero, rows_zero)],
            out_hbm.at[pl.ds(cid * n_tbl + sid * rows_zero, rows_zero)])

    return k


def _scatter_kernel(n, d, e_pad):
    ept = e_pad // _NW
    nchunks = ept // _CHUNK
    n_tbl = ((n + 16 + 127) // 128) * 128
    rows_zero = n_tbl // _NS

    nbuf = _NBUF

    rows_stage = n // _NS

    @functools.partial(
        pl.kernel,
        out_type=jax.ShapeDtypeStruct((_NC * n_tbl, d), jnp.float32),
        mesh=_sc_mesh(),
        compiler_params=pltpu.CompilerParams(use_tc_tiling_on_sc=False),
        scratch_types=[
            pltpu.VMEM((nchunks, _CHUNK), jnp.int32),
            pltpu.VMEM((nbuf, _CHUNK), jnp.int32),
        ] + [pltpu.VMEM((_CHUNK, d), jnp.float32)] * nbuf + [
            pltpu.VMEM_SHARED((n_tbl, d), jnp.float32),
            pltpu.VMEM_SHARED((n, d), jnp.float32),
            pltpu.SemaphoreType.DMA,
            pltpu.SemaphoreType.DMA,
        ],
    )
    def k(src_hbm, dst_hbm, g_hbm, zeros_hbm, out_hbm, dst_v, idxw, *rest):
        rows = rest[:nbuf]
        acc = rest[nbuf]
        g_sp = rest[nbuf + 1]
        sem_g = rest[nbuf + 2]
        sem_s = rest[nbuf + 3]
        cid = lax.axis_index("c")
        sid = lax.axis_index("s")
        wid = sid * _NC + cid
        pltpu.sync_copy(dst_hbm.at[pl.ds(wid * nchunks, nchunks)], dst_v)
        pltpu.sync_copy(
            g_hbm.at[pl.ds(sid * rows_stage, rows_stage)],
            g_sp.at[pl.ds(sid * rows_stage, rows_stage)])
        pltpu.sync_copy(zeros_hbm, acc.at[pl.ds(sid * rows_zero, rows_zero)])
        plsc.subcore_barrier()
        base = wid * nchunks

        def body(jo, _):
            pltpu.sync_copy(
                src_hbm.at[pl.ds(base + jo * nbuf, nbuf)], idxw)
            gdescs = [
                pltpu.async_copy(g_sp.at[idxw.at[b]], rows[b], sem_g)
                for b in range(nbuf)
            ]
            sdescs = []
            for b in range(nbuf):
                gdescs[b].wait()
                sdescs.append(pltpu.async_copy(
                    rows[b], acc.at[dst_v.at[jo * nbuf + b]],
                    sem_s, add=True))
            for dsc in sdescs:
                dsc.wait()
            return 0

        lax.fori_loop(0, nchunks // nbuf, body, 0)
        plsc.subcore_barrier()
        pltpu.sync_copy(
            acc.at[pl.ds(sid * rows_zero, rows_zero)],
            out_hbm.at[pl.ds(cid * n_tbl + sid * rows_zero, rows_zero)])

    return k


def _logits_kernel(n, e):
    per = e // _NW
    iters = per // 16

    @functools.partial(
        pl.kernel,
        out_type=jax.ShapeDtypeStruct((2 * e,), jnp.float32),
        mesh=_sc_mesh(),
        compiler_params=pltpu.CompilerParams(needs_layout_passes=False),
        scratch_types=[
            pltpu.VMEM((n,), jnp.float32),
            pltpu.VMEM((n,), jnp.float32),
            pltpu.VMEM((per,), jnp.int32),
            pltpu.VMEM((per,), jnp.int32),
            pltpu.VMEM((per,), jnp.float32),
        ],
    )
    def k(u_hbm, v_hbm, p0_hbm, p1_hbm, n0_hbm, n1_hbm, out_hbm,
          u_v, v_v, a_v, b_v, o_v):
        cid = lax.axis_index("c")
        sid = lax.axis_index("s")
        wid = sid * _NC + cid
        pltpu.sync_copy(u_hbm, u_v)
        pltpu.sync_copy(v_hbm, v_v)
        ebase = wid * per
        for a_hbm, b_hbm, obase in (
                (p0_hbm, p1_hbm, ebase),
                (n0_hbm, n1_hbm, e + ebase)):
            pltpu.sync_copy(a_hbm.at[pl.ds(ebase, per)], a_v)
            pltpu.sync_copy(b_hbm.at[pl.ds(ebase, per)], b_v)

            def body(i, _):
                ia = a_v[pl.ds(i * 16, 16)]
                ib = b_v[pl.ds(i * 16, 16)]
                ga = plsc.load_gather(u_v, [ia])
                gb = plsc.load_gather(v_v, [ib])
                o_v[pl.ds(i * 16, 16)] = ga + gb
                return 0

            lax.fori_loop(0, iters, body, 0)
            pltpu.sync_copy(o_v, out_hbm.at[pl.ds(obase, per)])

    return k


def _tc1(x, w1, degp, n, d_in, d_hid, bn):
    def body(x_ref, w_ref, degp_ref, dinv_ref, g1_ref):
        deg = degp_ref[0][:, 0:1] + degp_ref[1][:, 0:1] + 1.0
        dinv = lax.rsqrt(deg)
        dinv_ref[...] = dinv
        t = jnp.dot(x_ref[...], w_ref[...], preferred_element_type=jnp.float32)
        g1_ref[...] = t * dinv

    grid = n // bn
    return pl.pallas_call(
        body,
        grid=(grid,),
        in_specs=[
            pl.BlockSpec((bn, d_in), lambda i: (i, 0)),
            pl.BlockSpec((d_in, d_hid), lambda i: (0, 0)),
            pl.BlockSpec((_NC, bn, _DEGW), lambda i: (0, i, 0)),
        ],
        out_specs=[
            pl.BlockSpec((bn, 1), lambda i: (i, 0)),
            pl.BlockSpec((bn, d_hid), lambda i: (i, 0)),
        ],
        out_shape=[
            jax.ShapeDtypeStruct((n, 1), jnp.float32),
            jax.ShapeDtypeStruct((n, d_hid), jnp.float32),
        ],
    )(x, w1, degp)


def _tc2(p1, g1, dinv, b1, w2, n, d_hid, d_out, bn):
    def body(p_ref, g1_ref, dinv_ref, b1_ref, w_ref, g2_ref):
        s = p_ref[0] + p_ref[1] + g1_ref[...]
        h = jnp.maximum(s * dinv_ref[...] + b1_ref[...], 0.0)
        t = jnp.dot(h, w_ref[...], preferred_element_type=jnp.float32)
        g2_ref[...] = t * dinv_ref[...]

    grid = n // bn
    return pl.pallas_call(
        body,
        grid=(grid,),
        in_specs=[
            pl.BlockSpec((_NC, bn, d_hid), lambda i: (0, i, 0)),
            pl.BlockSpec((bn, d_hid), lambda i: (i, 0)),
            pl.BlockSpec((bn, 1), lambda i: (i, 0)),
            pl.BlockSpec((1, d_hid), lambda i: (0, 0)),
            pl.BlockSpec((d_hid, d_out), lambda i: (0, 0)),
        ],
        out_specs=pl.BlockSpec((bn, d_out), lambda i: (i, 0)),
        out_shape=jax.ShapeDtypeStruct((n, d_out), jnp.float32),
    )(p1, g1, dinv, b1, w2)


def _tc3(p2, g2, dinv, b2, wep_row, b_ep, n, d_out, bn):
    def body(p_ref, g2_ref, dinv_ref, b2_ref, w_ref, bep_ref,
             z_ref, u_ref, v_ref):
        s = p_ref[0] + p_ref[1] + g2_ref[...]
        z = s * dinv_ref[...] + b2_ref[...]
        z_ref[...] = z
        wa = w_ref[:, 0:d_out]
        wb = w_ref[:, d_out:2 * d_out]
        u_ref[...] = jnp.sum(z * wa, axis=1, keepdims=True) + bep_ref[0, 0]
        v_ref[...] = jnp.sum(z * wb, axis=1, keepdims=True)

    grid = n // bn
    return pl.pallas_call(
        body,
        grid=(grid,),
        in_specs=[
            pl.BlockSpec((_NC, bn, d_out), lambda i: (0, i, 0)),
            pl.BlockSpec((bn, d_out), lambda i: (i, 0)),
            pl.BlockSpec((bn, 1), lambda i: (i, 0)),
            pl.BlockSpec((1, d_out), lambda i: (0, 0)),
            pl.BlockSpec((1, 2 * d_out), lambda i: (0, 0)),
            pl.BlockSpec((1, 1), lambda i: (0, 0)),
        ],
        out_specs=[
            pl.BlockSpec((bn, d_out), lambda i: (i, 0)),
            pl.BlockSpec((bn, 1), lambda i: (i, 0)),
            pl.BlockSpec((bn, 1), lambda i: (i, 0)),
        ],
        out_shape=[
            jax.ShapeDtypeStruct((n, d_out), jnp.float32),
            jax.ShapeDtypeStruct((n, 1), jnp.float32),
            jax.ShapeDtypeStruct((n, 1), jnp.float32),
        ],
    )(p2, g2, dinv, b2, wep_row, b_ep)


def kernel(x, edge_index, pos_edge_index, neg_edge_index,
           W1, b1, W2, b2, W_ep, b_ep):
    n, d_in = x.shape
    e = edge_index.shape[1]
    d_hid = W1.shape[1]
    d_out = W2.shape[1]
    bn = 1000

    ept = -(-e // (_NW * _CHUNK * _NBUF)) * (_CHUNK * _NBUF)
    e_pad = ept * _NW
    pad = e_pad - e
    e_src = jnp.concatenate(
        [edge_index[0], jnp.zeros((pad,), jnp.int32)]).reshape(-1, _CHUNK)
    e_dst = jnp.concatenate(
        [edge_index[1], jnp.full((pad,), n, jnp.int32)]).reshape(-1, _CHUNK)

    ones_c = jnp.ones((_CHUNK, _DEGW), jnp.float32)
    n_tbl = ((n + 16 + 127) // 128) * 128
    zeros_deg = jnp.zeros((n_tbl // _NS, _DEGW), jnp.float32)
    zeros16 = jnp.zeros((n_tbl // _NS, d_hid), jnp.float32)
    zeros32 = jnp.zeros((n_tbl // _NS, d_out), jnp.float32)

    degp = _deg_kernel(n, e_pad)(e_dst, ones_c, zeros_deg)
    degp = degp.reshape(_NC, n_tbl, _DEGW)

    dinv, g1 = _tc1(x, W1, degp, n, d_in, d_hid, bn)

    p1 = _scatter_kernel(n, d_hid, e_pad)(e_src, e_dst, g1, zeros16)
    p1 = p1.reshape(_NC, n_tbl, d_hid)

    g2 = _tc2(p1, g1, dinv, b1.reshape(1, d_hid), W2, n, d_hid, d_out, bn)

    p2 = _scatter_kernel(n, d_out, e_pad)(e_src, e_dst, g2, zeros32)
    p2 = p2.reshape(_NC, n_tbl, d_out)

    z, u, v = _tc3(p2, g2, dinv, b2.reshape(1, d_out),
                   W_ep.reshape(1, 2 * d_out), b_ep.reshape(1, 1),
                   n, d_out, bn)

    logits = _logits_kernel(n, e)(
        u.reshape(n), v.reshape(n),
        pos_edge_index[0], pos_edge_index[1],
        neg_edge_index[0], neg_edge_index[1])

    return (z, logits.reshape(2 * e, 1))
